# Initial kernel scaffold; baseline (speedup 1.0000x reference)
import jax
import jax.numpy as jnp
from jax.experimental import pallas as pl


def kernel(params, origin_node, origin_edge, origin_edge_index, frag_node, frag_edge, frag_edge_index, frag_node2graph, motif_node, motif_edge, motif_edge_index, motif_node2graph):
    raise NotImplementedError("write your pallas kernel here")


# speedup vs baseline: 2.1402x; 2.1402x over previous
import functools

import jax
import jax.numpy as jnp
from jax.experimental import pallas as pl
from jax.experimental.pallas import tpu as pltpu
from jax.scipy.linalg import block_diag

D = 64
NEG = 0.01
NUM_GRAPHS = 250


def _lrelu(x):
    return jnp.where(x >= 0, x, NEG * x)


def _elu(x):
    return jnp.where(x > 0, x, jnp.exp(jnp.minimum(x, 0.0)) - 1.0)


def _sigmoid(x):
    return jax.nn.sigmoid(x)


def _pre_body(node_ref, We_ref, be_ref, W1_ref, b1_ref, wa1_ref, qb_ref,
              fn_ref, p_ref, q_ref):
    fn = _lrelu(jnp.dot(node_ref[...], We_ref[...],
                        preferred_element_type=jnp.float32) + be_ref[...])
    fn_ref[...] = fn
    p_ref[...] = jnp.dot(fn, W1_ref[...],
                         preferred_element_type=jnp.float32) + b1_ref[...]
    q_ref[...] = jnp.dot(fn, wa1_ref[...],
                         preferred_element_type=jnp.float32) + qb_ref[...]


def _pre_call(node, We, be, W1cat, b1cat, wa1cat, qbias, blk):
    n, din = node.shape
    grid = (n // blk,)
    return pl.pallas_call(
        _pre_body,
        grid=grid,
        in_specs=[
            pl.BlockSpec((blk, din), lambda i: (i, 0)),
            pl.BlockSpec(We.shape, lambda i: (0, 0)),
            pl.BlockSpec(be.shape, lambda i: (0, 0)),
            pl.BlockSpec(W1cat.shape, lambda i: (0, 0)),
            pl.BlockSpec(b1cat.shape, lambda i: (0, 0)),
            pl.BlockSpec(wa1cat.shape, lambda i: (0, 0)),
            pl.BlockSpec(qbias.shape, lambda i: (0, 0)),
        ],
        out_specs=[
            pl.BlockSpec((blk, D), lambda i: (i, 0)),
            pl.BlockSpec((blk, 2 * D), lambda i: (i, 0)),
            pl.BlockSpec((blk, 8), lambda i: (i, 0)),
        ],
        out_shape=[
            jax.ShapeDtypeStruct((n, D), jnp.float32),
            jax.ShapeDtypeStruct((n, 2 * D), jnp.float32),
            jax.ShapeDtypeStruct((n, 8), jnp.float32),
        ],
    )(node, We, be, W1cat, b1cat, wa1cat, qbias)


def _edge_body(ef_ref, gp_ref, gq_ref, We_ref, be_ref, W2_ref, wa2_ref,
               Wat_ref, bat_ref, t_ref, a_ref):
    fe = _lrelu(jnp.dot(ef_ref[...], We_ref[...],
                        preferred_element_type=jnp.float32) + be_ref[...])
    m = _lrelu(gp_ref[...] + jnp.dot(fe, W2_ref[...],
                                     preferred_element_type=jnp.float32))
    a_ref[...] = _lrelu(gq_ref[...] + jnp.dot(m, wa2_ref[...],
                                              preferred_element_type=jnp.float32))
    t_ref[...] = jnp.dot(m, Wat_ref[...],
                         preferred_element_type=jnp.float32) + bat_ref[...]


def _edge_call(ef, gp, gq, We, be, W2cat, wa2pad, Watblk, batcat, blk):
    e, dfe = ef.shape
    grid = (e // blk,)
    return pl.pallas_call(
        _edge_body,
        grid=grid,
        in_specs=[
            pl.BlockSpec((blk, dfe), lambda i: (i, 0)),
            pl.BlockSpec((blk, 2 * D), lambda i: (i, 0)),
            pl.BlockSpec((blk, 8), lambda i: (i, 0)),
            pl.BlockSpec(We.shape, lambda i: (0, 0)),
            pl.BlockSpec(be.shape, lambda i: (0, 0)),
            pl.BlockSpec(W2cat.shape, lambda i: (0, 0)),
            pl.BlockSpec(wa2pad.shape, lambda i: (0, 0)),
            pl.BlockSpec(Watblk.shape, lambda i: (0, 0)),
            pl.BlockSpec(batcat.shape, lambda i: (0, 0)),
        ],
        out_specs=[
            pl.BlockSpec((blk, 2 * D), lambda i: (i, 0)),
            pl.BlockSpec((blk, 8), lambda i: (i, 0)),
        ],
        out_shape=[
            jax.ShapeDtypeStruct((e, 2 * D), jnp.float32),
            jax.ShapeDtypeStruct((e, 8), jnp.float32),
        ],
    )(ef, gp, gq, We, be, W2cat, wa2pad, Watblk, batcat)


def _atomctx_body(num_ref, den_ref, h_ref, Wz_ref, Uz_ref, bz_ref, Wr_ref,
                  Ur_ref, br_ref, Wn_ref, Un_ref, bn_ref, wm2_ref, rb_ref,
                  Watm_ref, batm_ref, hn_ref, rm_ref, tm_ref):
    den = den_ref[...][:, 0:2] + 1e-16
    lane = jax.lax.broadcasted_iota(jnp.int32, num_ref.shape, 1)
    den = jnp.where(lane < D, den[:, 0:1], den[:, 1:2])
    ctx = _elu(num_ref[...] / den)
    h = h_ref[...]
    dot = lambda x, w: jnp.dot(x, w[...], preferred_element_type=jnp.float32)
    z = _sigmoid(dot(ctx, Wz_ref) + dot(h, Uz_ref) + bz_ref[...])
    r = _sigmoid(dot(ctx, Wr_ref) + dot(h, Ur_ref) + br_ref[...])
    n = jnp.tanh(dot(ctx, Wn_ref) + dot(r * h, Un_ref) + bn_ref[...])
    hn = jnp.maximum((1.0 - z) * n + z * h, 0.0)
    hn_ref[...] = hn
    rm_ref[...] = dot(hn, wm2_ref) + rb_ref[...]
    tm_ref[...] = dot(hn, Watm_ref) + batm_ref[...]


def _atomctx_call(num, den, h12, gru, wm2pad, rmbias, Watmblk, batmcat, blk):
    n = num.shape[0]
    grid = (n // blk,)
    full = lambda a: pl.BlockSpec(a.shape, lambda i: (0, 0))
    return pl.pallas_call(
        _atomctx_body,
        grid=grid,
        in_specs=[
            pl.BlockSpec((blk, 2 * D), lambda i: (i, 0)),
            pl.BlockSpec((blk, 8), lambda i: (i, 0)),
            pl.BlockSpec((blk, 2 * D), lambda i: (i, 0)),
            full(gru['Wz']), full(gru['Uz']), full(gru['bz']),
            full(gru['Wr']), full(gru['Ur']), full(gru['br']),
            full(gru['Wn']), full(gru['Un']), full(gru['bn']),
            full(wm2pad), full(rmbias), full(Watmblk), full(batmcat),
        ],
        out_specs=[
            pl.BlockSpec((blk, 2 * D), lambda i: (i, 0)),
            pl.BlockSpec((blk, 8), lambda i: (i, 0)),
            pl.BlockSpec((blk, 2 * D), lambda i: (i, 0)),
        ],
        out_shape=[
            jax.ShapeDtypeStruct((n, 2 * D), jnp.float32),
            jax.ShapeDtypeStruct((n, 8), jnp.float32),
            jax.ShapeDtypeStruct((n, 2 * D), jnp.float32),
        ],
    )(num, den, h12, gru['Wz'], gru['Uz'], gru['bz'], gru['Wr'], gru['Ur'],
      gru['br'], gru['Wn'], gru['Un'], gru['bn'], wm2pad, rmbias, Watmblk,
      batmcat)


def _molctx_body(num_ref, den_ref, s_ref, Wz_ref, Uz_ref, bz_ref, Wr_ref,
                 Ur_ref, br_ref, Wn_ref, Un_ref, bn_ref, sn_ref):
    den = den_ref[...][:, 0:2] + 1e-16
    lane = jax.lax.broadcasted_iota(jnp.int32, num_ref.shape, 1)
    den = jnp.where(lane < D, den[:, 0:1], den[:, 1:2])
    ctx = _elu(num_ref[...] / den)
    s = s_ref[...]
    dot = lambda x, w: jnp.dot(x, w[...], preferred_element_type=jnp.float32)
    z = _sigmoid(dot(ctx, Wz_ref) + dot(s, Uz_ref) + bz_ref[...])
    r = _sigmoid(dot(ctx, Wr_ref) + dot(s, Ur_ref) + br_ref[...])
    n = jnp.tanh(dot(ctx, Wn_ref) + dot(r * s, Un_ref) + bn_ref[...])
    sn_ref[...] = (1.0 - z) * n + z * s


def _molctx_call(num, den, s12, gru, blk):
    n = num.shape[0]
    grid = (n // blk,)
    full = lambda a: pl.BlockSpec(a.shape, lambda i: (0, 0))
    return pl.pallas_call(
        _molctx_body,
        grid=grid,
        in_specs=[
            pl.BlockSpec((blk, 2 * D), lambda i: (i, 0)),
            pl.BlockSpec((blk, 8), lambda i: (i, 0)),
            pl.BlockSpec((blk, 2 * D), lambda i: (i, 0)),
            full(gru['Wz']), full(gru['Uz']), full(gru['bz']),
            full(gru['Wr']), full(gru['Ur']), full(gru['br']),
            full(gru['Wn']), full(gru['Un']), full(gru['bn']),
        ],
        out_specs=pl.BlockSpec((blk, 2 * D), lambda i: (i, 0)),
        out_shape=jax.ShapeDtypeStruct((n, 2 * D), jnp.float32),
    )(num, den, s12, gru['Wz'], gru['Uz'], gru['bz'], gru['Wr'], gru['Ur'],
      gru['br'], gru['Wn'], gru['Un'], gru['bn'])


def _motifpre_body(mnd_ref, gm_ref, Wemb_ref, bemb_ref, P_ref, pb_ref,
                   W1_ref, b1_ref, wa1_ref, qb_ref, x_ref, p_ref, q_ref):
    dot = lambda x, w: jnp.dot(x, w[...], preferred_element_type=jnp.float32)
    mn = _lrelu(dot(mnd_ref[...], Wemb_ref) + bemb_ref[...])
    mnode = jnp.concatenate([gm_ref[...], mn], axis=1)
    x = dot(mnode, P_ref) + pb_ref[...]
    x_ref[...] = x
    p_ref[...] = dot(x, W1_ref) + b1_ref[...]
    q_ref[...] = dot(x, wa1_ref) + qb_ref[...]


def _motifpre_call(motif_node, gm, Wemb, bemb, Pcat, pbcat, W1blk, b1cat,
                   wa1blk, qbias, blk):
    n, din = motif_node.shape
    grid = (n // blk,)
    full = lambda a: pl.BlockSpec(a.shape, lambda i: (0, 0))
    return pl.pallas_call(
        _motifpre_body,
        grid=grid,
        in_specs=[
            pl.BlockSpec((blk, din), lambda i: (i, 0)),
            pl.BlockSpec((blk, D), lambda i: (i, 0)),
            full(Wemb), full(bemb), full(Pcat), full(pbcat),
            full(W1blk), full(b1cat), full(wa1blk), full(qbias),
        ],
        out_specs=[
            pl.BlockSpec((blk, 2 * D), lambda i: (i, 0)),
            pl.BlockSpec((blk, 2 * D), lambda i: (i, 0)),
            pl.BlockSpec((blk, 8), lambda i: (i, 0)),
        ],
        out_shape=[
            jax.ShapeDtypeStruct((n, 2 * D), jnp.float32),
            jax.ShapeDtypeStruct((n, 2 * D), jnp.float32),
            jax.ShapeDtypeStruct((n, 8), jnp.float32),
        ],
    )(motif_node, gm, Wemb, bemb, Pcat, pbcat, W1blk, b1cat, wa1blk, qbias)


def _gm_body(sn_ref, Wfa_ref, bfa_ref, gm_ref):
    gm_ref[...] = jnp.maximum(
        jnp.dot(sn_ref[...], Wfa_ref[...],
                preferred_element_type=jnp.float32) + bfa_ref[...], 0.0)


def _gm_call(sn12, Wfa, bfa, blk):
    n = sn12.shape[0]
    grid = (n // blk,)
    return pl.pallas_call(
        _gm_body,
        grid=grid,
        in_specs=[
            pl.BlockSpec((blk, 2 * D), lambda i: (i, 0)),
            pl.BlockSpec(Wfa.shape, lambda i: (0, 0)),
            pl.BlockSpec(bfa.shape, lambda i: (0, 0)),
        ],
        out_specs=pl.BlockSpec((blk, D), lambda i: (i, 0)),
        out_shape=jax.ShapeDtypeStruct((n, D), jnp.float32),
    )(sn12, Wfa, bfa)


def _final_body(sn_ref, W1_ref, b1_ref, W2_ref, b2_ref, out_ref):
    sn = sn_ref[...]
    sup = jnp.maximum(0.5 * (sn[:, 0:D] + sn[:, D:2 * D]), 0.0)
    h1 = _lrelu(jnp.dot(sup, W1_ref[...],
                        preferred_element_type=jnp.float32) + b1_ref[...])
    out_ref[...] = jnp.dot(h1, W2_ref[...],
                           preferred_element_type=jnp.float32) + b2_ref[...]


def _final_call(sn12, W1, b1, W2, b2):
    n = sn12.shape[0]
    return pl.pallas_call(
        _final_body,
        in_specs=[pl.BlockSpec(sn12.shape, lambda: (0, 0)),
                  pl.BlockSpec(W1.shape, lambda: (0, 0)),
                  pl.BlockSpec(b1.shape, lambda: (0, 0)),
                  pl.BlockSpec(W2.shape, lambda: (0, 0)),
                  pl.BlockSpec(b2.shape, lambda: (0, 0))],
        out_specs=pl.BlockSpec((n, 1), lambda: (0, 0)),
        out_shape=jax.ShapeDtypeStruct((n, 1), jnp.float32),
    )(sn12, W1, b1, W2, b2)


def _pack_atom(heads):
    W1cat = jnp.concatenate([h['ne']['W'][:D] for h in heads], axis=1)
    b1cat = jnp.concatenate([h['ne']['b'] for h in heads])[None, :]
    W2cat = jnp.concatenate([h['ne']['W'][D:] for h in heads], axis=1)
    wa1 = jnp.concatenate([h['al']['W'][:D] for h in heads], axis=1)
    wa1cat = jnp.pad(wa1, ((0, 0), (0, 6)))
    qb = jnp.array([heads[0]['al']['b'][0], heads[1]['al']['b'][0],
                    1.0, 0, 0, 0, 0, 0], jnp.float32)[None, :]
    wa2 = block_diag(heads[0]['al']['W'][D:], heads[1]['al']['W'][D:])
    wa2pad = jnp.pad(wa2, ((0, 0), (0, 6)))
    Watblk = block_diag(heads[0]['at']['W'], heads[1]['at']['W'])
    batcat = jnp.concatenate([h['at']['b'] for h in heads])[None, :]
    gru = {k: block_diag(heads[0]['gru'][k], heads[1]['gru'][k])
           if heads[0]['gru'][k].ndim == 2
           else jnp.concatenate([h['gru'][k] for h in heads])[None, :]
           for k in heads[0]['gru']}
    return dict(W1cat=W1cat, b1cat=b1cat, W2cat=W2cat, wa1cat=wa1cat, qb=qb,
                wa2pad=wa2pad, Watblk=Watblk, batcat=batcat, gru=gru)


def _pack_mol(heads):
    wm1blk = jnp.pad(block_diag(*[h['al']['W'][:D] for h in heads]),
                     ((0, 0), (0, 6)))
    qb = jnp.array([heads[0]['al']['b'][0], heads[1]['al']['b'][0],
                    1.0, 0, 0, 0, 0, 0], jnp.float32)[None, :]
    wm2 = block_diag(*[h['al']['W'][D:] for h in heads])
    wm2pad = jnp.pad(wm2, ((0, 0), (0, 6)))
    rmb = jnp.zeros((1, 8), jnp.float32)
    Watblk = block_diag(*[h['at']['W'] for h in heads])
    batcat = jnp.concatenate([h['at']['b'] for h in heads])[None, :]
    gru = {k: block_diag(*[h['gru'][k] for h in heads])
           if heads[0]['gru'][k].ndim == 2
           else jnp.concatenate([h['gru'][k] for h in heads])[None, :]
           for k in heads[0]['gru']}
    return dict(wm1blk=wm1blk, qb=qb, wm2pad=wm2pad, rmb=rmb, Watblk=Watblk,
                batcat=batcat, gru=gru)


def _blk(n, target):
    if n <= target:
        return n
    k = -(-n // target)
    while n % k:
        k += 1
    return n // k


def _gather_rows(tab, idx):
    return jnp.take(tab, idx, axis=0)


def _segsum(x, seg, n):
    return jax.ops.segment_sum(x, seg, num_segments=n)


def _softmax_stage(t, a8, seg, nseg):
    suma = _segsum(a8, seg, nseg)
    cnt = jnp.maximum(suma[:, 2:3], 1.0)
    amean = suma[:, 0:2] / cnt
    ex = jnp.exp(a8[:, 0:2] - amean[seg])
    den = _segsum(jnp.pad(ex, ((0, 0), (0, 6))), seg, nseg)
    exw = jnp.concatenate([jnp.repeat(ex[:, 0:1], D, 1),
                           jnp.repeat(ex[:, 1:2], D, 1)], axis=1)
    num = _segsum(exw * t, seg, nseg)
    return num, den


def _atom_stage(pk, ef, src, dst, p12, q12, h12, n_nodes, molpk, eblk, nblk):
    gp = _gather_rows(p12, src)
    gq = _gather_rows(q12, dst)
    t, a8 = _edge_call(ef, gp, gq, pk['We'], pk['be'], pk['W2cat'],
                       pk['wa2pad'], pk['Watblk'], pk['batcat'], eblk)
    num, den = _softmax_stage(t, a8, dst, n_nodes)
    return _atomctx_call(num, den, h12, pk['gru'], molpk['wm2pad'],
                         molpk['rmb'], molpk['Watblk'], molpk['batcat'], nblk)


def _mol_stage(molpk, h12, rm12, tm12, n2g, nseg, gru_blk, nblk):
    s12 = _segsum(h12, n2g, nseg)
    qm = jnp.dot(s12, molpk['wm1blk']) + molpk['qb']
    a8 = _lrelu(_gather_rows(qm, n2g) + rm12)
    num, den = _softmax_stage(tm12, a8, n2g, nseg)
    return _molctx_call(num, den, s12, molpk['gru'], nblk)


def kernel(params, origin_node, origin_edge, origin_edge_index, frag_node,
           frag_edge, frag_edge_index, frag_node2graph, motif_node,
           motif_edge, motif_edge_index, motif_node2graph):
    V = frag_node.shape[0]
    E = frag_edge.shape[0]
    NF = motif_node.shape[0]
    EM = motif_edge.shape[0]
    G = NUM_GRAPHS

    fsrc = jnp.asarray(frag_edge_index[0], jnp.int32)
    fdst = jnp.asarray(frag_edge_index[1], jnp.int32)
    msrc = jnp.asarray(motif_edge_index[0], jnp.int32)
    mdst = jnp.asarray(motif_edge_index[1], jnp.int32)
    fn2g = jnp.asarray(frag_node2graph, jnp.int32)
    mn2g = jnp.asarray(motif_node2graph, jnp.int32)

    fh = params['frag_heads']
    jh = params['junc_heads']

    fa = _pack_atom([h['atom'] for h in fh])
    fa['We'] = params['emb_fe']['W']
    fa['be'] = params['emb_fe']['b'][None, :]
    fm = _pack_mol([h['mol'] for h in fh])
    ja = _pack_atom([h['atom'] for h in jh])
    ja['We'] = params['emb_me']['W']
    ja['be'] = params['emb_me']['b'][None, :]
    jm = _pack_mol([h['mol'] for h in jh])

    fn, p12, q12 = _pre_call(
        frag_node, params['emb_fn']['W'], params['emb_fn']['b'][None, :],
        fa['W1cat'], fa['b1cat'], fa['wa1cat'], fa['qb'], _blk(V, 2000))
    h12 = jnp.concatenate([fn, fn], axis=1)
    hn12, rm12, tm12 = _atom_stage(fa, frag_edge, fsrc, fdst, p12, q12, h12,
                                   V, fm, _blk(E, 2000), _blk(V, 2000))
    sf12 = _mol_stage(fm, hn12, rm12, tm12, fn2g, NF, None, _blk(NF, 2500))
    gm = _gm_call(sf12, params['frag_attend']['W'],
                  params['frag_attend']['b'][None, :], _blk(NF, 2500))

    Pcat = jnp.concatenate([h['proj']['W'] for h in jh], axis=1)
    pbcat = jnp.concatenate([h['proj']['b'] for h in jh])[None, :]
    W1blk = block_diag(*[h['atom']['ne']['W'][:D] for h in jh])
    b1cat = jnp.concatenate([h['atom']['ne']['b'] for h in jh])[None, :]
    wa1blk = jnp.pad(block_diag(*[h['atom']['al']['W'][:D] for h in jh]),
                     ((0, 0), (0, 6)))
    jqb = jnp.array([jh[0]['atom']['al']['b'][0], jh[1]['atom']['al']['b'][0],
                     1.0, 0, 0, 0, 0, 0], jnp.float32)[None, :]
    x12, pj12, qj12 = _motifpre_call(motif_node, gm, params['emb_mn']['W'],
                                     params['emb_mn']['b'][None, :], Pcat,
                                     pbcat, W1blk, b1cat, wa1blk, jqb,
                                     _blk(NF, 2500))

    xn12, rj12, tj12 = _atom_stage(ja, motif_edge, msrc, mdst, pj12, qj12,
                                   x12, NF, jm, _blk(EM, 2000), _blk(NF, 2500))
    sg12 = _mol_stage(jm, xn12, rj12, tj12, mn2g, G, None, G)

    return _final_call(sg12, params['pred1']['W'],
                       params['pred1']['b'][None, :], params['pred2']['W'],
                       params['pred2']['b'][None, :])


# speedup vs baseline: 2.7953x; 1.3061x over previous
import functools

import jax
import jax.numpy as jnp
from jax import lax
from jax.experimental import pallas as pl
from jax.experimental.pallas import tpu as pltpu
from jax.experimental.pallas import tpu_sc as plsc
from jax.scipy.linalg import block_diag

D = 64
NEG = 0.01
NUM_GRAPHS = 250


def _lrelu(x):
    return jnp.where(x >= 0, x, NEG * x)


def _elu(x):
    return jnp.where(x > 0, x, jnp.exp(jnp.minimum(x, 0.0)) - 1.0)


def _sigmoid(x):
    return jax.nn.sigmoid(x)


def _pre_body(node_ref, We_ref, be_ref, W1_ref, b1_ref, wa1_ref, qb_ref,
              fn_ref, p_ref, q_ref):
    fn = _lrelu(jnp.dot(node_ref[...], We_ref[...],
                        preferred_element_type=jnp.float32) + be_ref[...])
    fn_ref[...] = fn
    p_ref[...] = jnp.dot(fn, W1_ref[...],
                         preferred_element_type=jnp.float32) + b1_ref[...]
    q_ref[...] = jnp.dot(fn, wa1_ref[...],
                         preferred_element_type=jnp.float32) + qb_ref[...]


def _pre_call(node, We, be, W1cat, b1cat, wa1cat, qbias, blk):
    n, din = node.shape
    grid = (n // blk,)
    return pl.pallas_call(
        _pre_body,
        grid=grid,
        in_specs=[
            pl.BlockSpec((blk, din), lambda i: (i, 0)),
            pl.BlockSpec(We.shape, lambda i: (0, 0)),
            pl.BlockSpec(be.shape, lambda i: (0, 0)),
            pl.BlockSpec(W1cat.shape, lambda i: (0, 0)),
            pl.BlockSpec(b1cat.shape, lambda i: (0, 0)),
            pl.BlockSpec(wa1cat.shape, lambda i: (0, 0)),
            pl.BlockSpec(qbias.shape, lambda i: (0, 0)),
        ],
        out_specs=[
            pl.BlockSpec((blk, D), lambda i: (i, 0)),
            pl.BlockSpec((blk, 2 * D), lambda i: (i, 0)),
            pl.BlockSpec((blk, 8), lambda i: (i, 0)),
        ],
        out_shape=[
            jax.ShapeDtypeStruct((n, D), jnp.float32),
            jax.ShapeDtypeStruct((n, 2 * D), jnp.float32),
            jax.ShapeDtypeStruct((n, 8), jnp.float32),
        ],
    )(node, We, be, W1cat, b1cat, wa1cat, qbias)


def _edge_body(ef_ref, gp_ref, gq_ref, We_ref, be_ref, W2_ref, wa2_ref,
               Wat_ref, bat_ref, t_ref, a_ref):
    fe = _lrelu(jnp.dot(ef_ref[...], We_ref[...],
                        preferred_element_type=jnp.float32) + be_ref[...])
    m = _lrelu(gp_ref[...] + jnp.dot(fe, W2_ref[...],
                                     preferred_element_type=jnp.float32))
    a_ref[...] = _lrelu(gq_ref[...] + jnp.dot(m, wa2_ref[...],
                                              preferred_element_type=jnp.float32))
    t_ref[...] = jnp.dot(m, Wat_ref[...],
                         preferred_element_type=jnp.float32) + bat_ref[...]


def _edge_call(ef, gp, gq, We, be, W2cat, wa2pad, Watblk, batcat, blk):
    e, dfe = ef.shape
    grid = (e // blk,)
    return pl.pallas_call(
        _edge_body,
        grid=grid,
        in_specs=[
            pl.BlockSpec((blk, dfe), lambda i: (i, 0)),
            pl.BlockSpec((blk, 2 * D), lambda i: (i, 0)),
            pl.BlockSpec((blk, 8), lambda i: (i, 0)),
            pl.BlockSpec(We.shape, lambda i: (0, 0)),
            pl.BlockSpec(be.shape, lambda i: (0, 0)),
            pl.BlockSpec(W2cat.shape, lambda i: (0, 0)),
            pl.BlockSpec(wa2pad.shape, lambda i: (0, 0)),
            pl.BlockSpec(Watblk.shape, lambda i: (0, 0)),
            pl.BlockSpec(batcat.shape, lambda i: (0, 0)),
        ],
        out_specs=[
            pl.BlockSpec((blk, 2 * D), lambda i: (i, 0)),
            pl.BlockSpec((blk, 8), lambda i: (i, 0)),
        ],
        out_shape=[
            jax.ShapeDtypeStruct((e, 2 * D), jnp.float32),
            jax.ShapeDtypeStruct((e, 8), jnp.float32),
        ],
    )(ef, gp, gq, We, be, W2cat, wa2pad, Watblk, batcat)


def _atomctx_body(num_ref, den_ref, h_ref, Wz_ref, Uz_ref, bz_ref, Wr_ref,
                  Ur_ref, br_ref, Wn_ref, Un_ref, bn_ref, wm2_ref, rb_ref,
                  Watm_ref, batm_ref, hn_ref, rm_ref, tm_ref):
    num = num_ref[0] + num_ref[1]
    den = (den_ref[0] + den_ref[1])[:, 0:2] + 1e-16
    lane = jax.lax.broadcasted_iota(jnp.int32, num.shape, 1)
    den = jnp.where(lane < D, den[:, 0:1], den[:, 1:2])
    ctx = _elu(num / den)
    h = h_ref[...]
    dot = lambda x, w: jnp.dot(x, w[...], preferred_element_type=jnp.float32)
    z = _sigmoid(dot(ctx, Wz_ref) + dot(h, Uz_ref) + bz_ref[...])
    r = _sigmoid(dot(ctx, Wr_ref) + dot(h, Ur_ref) + br_ref[...])
    n = jnp.tanh(dot(ctx, Wn_ref) + dot(r * h, Un_ref) + bn_ref[...])
    hn = jnp.maximum((1.0 - z) * n + z * h, 0.0)
    hn_ref[...] = hn
    rm_ref[...] = dot(hn, wm2_ref) + rb_ref[...]
    tm_ref[...] = dot(hn, Watm_ref) + batm_ref[...]


def _atomctx_call(num, den, h12, gru, wm2pad, rmbias, Watmblk, batmcat, blk):
    n = num.shape[1]
    grid = (n // blk,)
    full = lambda a: pl.BlockSpec(a.shape, lambda i: (0, 0))
    return pl.pallas_call(
        _atomctx_body,
        grid=grid,
        in_specs=[
            pl.BlockSpec((2, blk, 2 * D), lambda i: (0, i, 0)),
            pl.BlockSpec((2, blk, 8), lambda i: (0, i, 0)),
            pl.BlockSpec((blk, 2 * D), lambda i: (i, 0)),
            full(gru['Wz']), full(gru['Uz']), full(gru['bz']),
            full(gru['Wr']), full(gru['Ur']), full(gru['br']),
            full(gru['Wn']), full(gru['Un']), full(gru['bn']),
            full(wm2pad), full(rmbias), full(Watmblk), full(batmcat),
        ],
        out_specs=[
            pl.BlockSpec((blk, 2 * D), lambda i: (i, 0)),
            pl.BlockSpec((blk, 8), lambda i: (i, 0)),
            pl.BlockSpec((blk, 2 * D), lambda i: (i, 0)),
        ],
        out_shape=[
            jax.ShapeDtypeStruct((n, 2 * D), jnp.float32),
            jax.ShapeDtypeStruct((n, 8), jnp.float32),
            jax.ShapeDtypeStruct((n, 2 * D), jnp.float32),
        ],
    )(num, den, h12, gru['Wz'], gru['Uz'], gru['bz'], gru['Wr'], gru['Ur'],
      gru['br'], gru['Wn'], gru['Un'], gru['bn'], wm2pad, rmbias, Watmblk,
      batmcat)


def _molctx_body(num_ref, den_ref, s_ref, Wz_ref, Uz_ref, bz_ref, Wr_ref,
                 Ur_ref, br_ref, Wn_ref, Un_ref, bn_ref, sn_ref):
    num = num_ref[0] + num_ref[1]
    den = (den_ref[0] + den_ref[1])[:, 0:2] + 1e-16
    lane = jax.lax.broadcasted_iota(jnp.int32, num.shape, 1)
    den = jnp.where(lane < D, den[:, 0:1], den[:, 1:2])
    ctx = _elu(num / den)
    s = s_ref[...]
    dot = lambda x, w: jnp.dot(x, w[...], preferred_element_type=jnp.float32)
    z = _sigmoid(dot(ctx, Wz_ref) + dot(s, Uz_ref) + bz_ref[...])
    r = _sigmoid(dot(ctx, Wr_ref) + dot(s, Ur_ref) + br_ref[...])
    n = jnp.tanh(dot(ctx, Wn_ref) + dot(r * s, Un_ref) + bn_ref[...])
    sn_ref[...] = (1.0 - z) * n + z * s


def _molctx_call(num, den, s12, gru, blk):
    n = num.shape[1]
    grid = (n // blk,)
    full = lambda a: pl.BlockSpec(a.shape, lambda i: (0, 0))
    return pl.pallas_call(
        _molctx_body,
        grid=grid,
        in_specs=[
            pl.BlockSpec((2, blk, 2 * D), lambda i: (0, i, 0)),
            pl.BlockSpec((2, blk, 8), lambda i: (0, i, 0)),
            pl.BlockSpec((blk, 2 * D), lambda i: (i, 0)),
            full(gru['Wz']), full(gru['Uz']), full(gru['bz']),
            full(gru['Wr']), full(gru['Ur']), full(gru['br']),
            full(gru['Wn']), full(gru['Un']), full(gru['bn']),
        ],
        out_specs=pl.BlockSpec((blk, 2 * D), lambda i: (i, 0)),
        out_shape=jax.ShapeDtypeStruct((n, 2 * D), jnp.float32),
    )(num, den, s12, gru['Wz'], gru['Uz'], gru['bz'], gru['Wr'], gru['Ur'],
      gru['br'], gru['Wn'], gru['Un'], gru['bn'])


def _motifpre_body(mnd_ref, gm_ref, Wemb_ref, bemb_ref, P_ref, pb_ref,
                   W1_ref, b1_ref, wa1_ref, qb_ref, x_ref, p_ref, q_ref):
    dot = lambda x, w: jnp.dot(x, w[...], preferred_element_type=jnp.float32)
    mn = _lrelu(dot(mnd_ref[...], Wemb_ref) + bemb_ref[...])
    mnode = jnp.concatenate([gm_ref[...], mn], axis=1)
    x = dot(mnode, P_ref) + pb_ref[...]
    x_ref[...] = x
    p_ref[...] = dot(x, W1_ref) + b1_ref[...]
    q_ref[...] = dot(x, wa1_ref) + qb_ref[...]


def _motifpre_call(motif_node, gm, Wemb, bemb, Pcat, pbcat, W1blk, b1cat,
                   wa1blk, qbias, blk):
    n, din = motif_node.shape
    grid = (n // blk,)
    full = lambda a: pl.BlockSpec(a.shape, lambda i: (0, 0))
    return pl.pallas_call(
        _motifpre_body,
        grid=grid,
        in_specs=[
            pl.BlockSpec((blk, din), lambda i: (i, 0)),
            pl.BlockSpec((blk, D), lambda i: (i, 0)),
            full(Wemb), full(bemb), full(Pcat), full(pbcat),
            full(W1blk), full(b1cat), full(wa1blk), full(qbias),
        ],
        out_specs=[
            pl.BlockSpec((blk, 2 * D), lambda i: (i, 0)),
            pl.BlockSpec((blk, 2 * D), lambda i: (i, 0)),
            pl.BlockSpec((blk, 8), lambda i: (i, 0)),
        ],
        out_shape=[
            jax.ShapeDtypeStruct((n, 2 * D), jnp.float32),
            jax.ShapeDtypeStruct((n, 2 * D), jnp.float32),
            jax.ShapeDtypeStruct((n, 8), jnp.float32),
        ],
    )(motif_node, gm, Wemb, bemb, Pcat, pbcat, W1blk, b1cat, wa1blk, qbias)


def _gm_body(sn_ref, Wfa_ref, bfa_ref, gm_ref):
    gm_ref[...] = jnp.maximum(
        jnp.dot(sn_ref[...], Wfa_ref[...],
                preferred_element_type=jnp.float32) + bfa_ref[...], 0.0)


def _gm_call(sn12, Wfa, bfa, blk):
    n = sn12.shape[0]
    grid = (n // blk,)
    return pl.pallas_call(
        _gm_body,
        grid=grid,
        in_specs=[
            pl.BlockSpec((blk, 2 * D), lambda i: (i, 0)),
            pl.BlockSpec(Wfa.shape, lambda i: (0, 0)),
            pl.BlockSpec(bfa.shape, lambda i: (0, 0)),
        ],
        out_specs=pl.BlockSpec((blk, D), lambda i: (i, 0)),
        out_shape=jax.ShapeDtypeStruct((n, D), jnp.float32),
    )(sn12, Wfa, bfa)


def _final_body(sn_ref, W1_ref, b1_ref, W2_ref, b2_ref, out_ref):
    sn = sn_ref[...]
    sup = jnp.maximum(0.5 * (sn[:, 0:D] + sn[:, D:2 * D]), 0.0)
    h1 = _lrelu(jnp.dot(sup, W1_ref[...],
                        preferred_element_type=jnp.float32) + b1_ref[...])
    out_ref[...] = jnp.dot(h1, W2_ref[...],
                           preferred_element_type=jnp.float32) + b2_ref[...]


def _final_call(sn12, W1, b1, W2, b2):
    n = sn12.shape[0]
    return pl.pallas_call(
        _final_body,
        in_specs=[pl.BlockSpec(sn12.shape, lambda: (0, 0)),
                  pl.BlockSpec(W1.shape, lambda: (0, 0)),
                  pl.BlockSpec(b1.shape, lambda: (0, 0)),
                  pl.BlockSpec(W2.shape, lambda: (0, 0)),
                  pl.BlockSpec(b2.shape, lambda: (0, 0))],
        out_specs=pl.BlockSpec((n, 1), lambda: (0, 0)),
        out_shape=jax.ShapeDtypeStruct((n, 1), jnp.float32),
    )(sn12, W1, b1, W2, b2)


def _pack_atom(heads):
    W1cat = jnp.concatenate([h['ne']['W'][:D] for h in heads], axis=1)
    b1cat = jnp.concatenate([h['ne']['b'] for h in heads])[None, :]
    W2cat = jnp.concatenate([h['ne']['W'][D:] for h in heads], axis=1)
    wa1 = jnp.concatenate([h['al']['W'][:D] for h in heads], axis=1)
    wa1cat = jnp.pad(wa1, ((0, 0), (0, 6)))
    qb = jnp.array([heads[0]['al']['b'][0], heads[1]['al']['b'][0],
                    1.0, 0, 0, 0, 0, 0], jnp.float32)[None, :]
    wa2 = block_diag(heads[0]['al']['W'][D:], heads[1]['al']['W'][D:])
    wa2pad = jnp.pad(wa2, ((0, 0), (0, 6)))
    Watblk = block_diag(heads[0]['at']['W'], heads[1]['at']['W'])
    batcat = jnp.concatenate([h['at']['b'] for h in heads])[None, :]
    gru = {k: block_diag(heads[0]['gru'][k], heads[1]['gru'][k])
           if heads[0]['gru'][k].ndim == 2
           else jnp.concatenate([h['gru'][k] for h in heads])[None, :]
           for k in heads[0]['gru']}
    return dict(W1cat=W1cat, b1cat=b1cat, W2cat=W2cat, wa1cat=wa1cat, qb=qb,
                wa2pad=wa2pad, Watblk=Watblk, batcat=batcat, gru=gru)


def _pack_mol(heads):
    wm1blk = jnp.pad(block_diag(*[h['al']['W'][:D] for h in heads]),
                     ((0, 0), (0, 6)))
    qb = jnp.array([heads[0]['al']['b'][0], heads[1]['al']['b'][0],
                    1.0, 0, 0, 0, 0, 0], jnp.float32)[None, :]
    wm2 = block_diag(*[h['al']['W'][D:] for h in heads])
    wm2pad = jnp.pad(wm2, ((0, 0), (0, 6)))
    rmb = jnp.zeros((1, 8), jnp.float32)
    Watblk = block_diag(*[h['at']['W'] for h in heads])
    batcat = jnp.concatenate([h['at']['b'] for h in heads])[None, :]
    gru = {k: block_diag(*[h['gru'][k] for h in heads])
           if heads[0]['gru'][k].ndim == 2
           else jnp.concatenate([h['gru'][k] for h in heads])[None, :]
           for k in heads[0]['gru']}
    return dict(wm1blk=wm1blk, qb=qb, wm2pad=wm2pad, rmb=rmb, Watblk=Watblk,
                batcat=batcat, gru=gru)


def _amean_body(sp_ref, out_ref):
    s = sp_ref[0] + sp_ref[1]
    cnt = jnp.maximum(s[:, 2:3], 1.0)
    lane = jax.lax.broadcasted_iota(jnp.int32, s.shape, 1)
    out_ref[...] = jnp.where(lane < 2, s / cnt, 0.0)


def _amean_call(sparts, blk):
    n = sparts.shape[1]
    grid = (n // blk,)
    return pl.pallas_call(
        _amean_body,
        grid=grid,
        in_specs=[pl.BlockSpec((2, blk, 8), lambda i: (0, i, 0))],
        out_specs=pl.BlockSpec((blk, 8), lambda i: (i, 0)),
        out_shape=jax.ShapeDtypeStruct((n, 8), jnp.float32),
    )(sparts)


def _scale_body(t_ref, a_ref, gam_ref, ext_ref, ex8_ref):
    ex = jnp.exp(a_ref[...][:, 0:2] - gam_ref[...][:, 0:2])
    t = t_ref[...]
    lane = jax.lax.broadcasted_iota(jnp.int32, t.shape, 1)
    ext_ref[...] = t * jnp.where(lane < D, ex[:, 0:1], ex[:, 1:2])
    lane8 = jax.lax.broadcasted_iota(jnp.int32, (t.shape[0], 8), 1)
    ex8_ref[...] = jnp.where(lane8 == 0, ex[:, 0:1],
                             jnp.where(lane8 == 1, ex[:, 1:2], 0.0))


def _scale_call(t, a8, gam, blk):
    n = t.shape[0]
    grid = (n // blk,)
    return pl.pallas_call(
        _scale_body,
        grid=grid,
        in_specs=[
            pl.BlockSpec((blk, 2 * D), lambda i: (i, 0)),
            pl.BlockSpec((blk, 8), lambda i: (i, 0)),
            pl.BlockSpec((blk, 8), lambda i: (i, 0)),
        ],
        out_specs=[
            pl.BlockSpec((blk, 2 * D), lambda i: (i, 0)),
            pl.BlockSpec((blk, 8), lambda i: (i, 0)),
        ],
        out_shape=[
            jax.ShapeDtypeStruct((n, 2 * D), jnp.float32),
            jax.ShapeDtypeStruct((n, 8), jnp.float32),
        ],
    )(t, a8, gam)


def _a8_body(gqm_ref, rm_ref, out_ref):
    out_ref[...] = _lrelu(gqm_ref[...] + rm_ref[...])


def _a8_call(gqm, rm, blk):
    n = gqm.shape[0]
    grid = (n // blk,)
    return pl.pallas_call(
        _a8_body,
        grid=grid,
        in_specs=[pl.BlockSpec((blk, 8), lambda i: (i, 0)),
                  pl.BlockSpec((blk, 8), lambda i: (i, 0))],
        out_specs=pl.BlockSpec((blk, 8), lambda i: (i, 0)),
        out_shape=jax.ShapeDtypeStruct((n, 8), jnp.float32),
    )(gqm, rm)


def _molqm_body(sp_ref, wm1_ref, qb_ref, s_ref, qm_ref):
    s = sp_ref[0] + sp_ref[1]
    s_ref[...] = s
    qm_ref[...] = jnp.dot(s, wm1_ref[...],
                          preferred_element_type=jnp.float32) + qb_ref[...]


def _molqm_call(sparts, wm1blk, qb, blk):
    n = sparts.shape[1]
    grid = (n // blk,)
    return pl.pallas_call(
        _molqm_body,
        grid=grid,
        in_specs=[
            pl.BlockSpec((2, blk, 2 * D), lambda i: (0, i, 0)),
            pl.BlockSpec(wm1blk.shape, lambda i: (0, 0)),
            pl.BlockSpec(qb.shape, lambda i: (0, 0)),
        ],
        out_specs=[
            pl.BlockSpec((blk, 2 * D), lambda i: (i, 0)),
            pl.BlockSpec((blk, 8), lambda i: (i, 0)),
        ],
        out_shape=[
            jax.ShapeDtypeStruct((n, 2 * D), jnp.float32),
            jax.ShapeDtypeStruct((n, 8), jnp.float32),
        ],
    )(sparts, wm1blk, qb)


_NC, _NS, _NW = 2, 16, 32
_IB = 128
_CH = 256


def _sc_mesh():
    return plsc.VectorSubcoreMesh(core_axis_name="c", subcore_axis_name="s")


def _sc_gather(tab, idx2d, W):
    nrows = idx2d.shape[0] * _IB
    chunks = nrows // _CH
    nk = -(-chunks // _NW)

    @functools.partial(
        pl.kernel,
        out_type=jax.ShapeDtypeStruct((nrows, W), jnp.float32),
        mesh=_sc_mesh(),
        scratch_types=[
            pltpu.VMEM((_CH // _IB, _IB), jnp.int32),
            pltpu.VMEM((_CH, W), jnp.float32),
            pltpu.SemaphoreType.DMA,
        ],
    )
    def k(tab_hbm, idx_hbm, out_hbm, idx_v, rows_v, sem):
        wid = lax.axis_index("s") * _NC + lax.axis_index("c")

        def body(j, carry):
            c = wid + j * _NW

            @pl.when(c < chunks)
            def _():
                pltpu.sync_copy(idx_hbm.at[pl.ds(c * (_CH // _IB), _CH // _IB)],
                                idx_v)
                cps = [pltpu.async_copy(tab_hbm.at[idx_v.at[i]],
                                        rows_v.at[pl.ds(i * _IB, _IB)], sem)
                       for i in range(_CH // _IB)]
                for cp in cps:
                    cp.wait()
                pltpu.sync_copy(rows_v, out_hbm.at[pl.ds(c * _CH, _CH)])

            return carry

        lax.fori_loop(0, nk, body, 0)

    return k(tab, idx2d)


def _sc_scatter(rows_list, idx2d, N):
    nrows = idx2d.shape[0] * _IB
    chunks = nrows // _CH
    nk = -(-chunks // _NW)
    Ws = [r.shape[1] for r in rows_list]
    rows_pt = N // _NS

    out_type = [jax.ShapeDtypeStruct((_NC, N, W), jnp.float32) for W in Ws]
    scratch = [pltpu.VMEM((_CH // _IB, _IB), jnp.int32)]
    scratch += [pltpu.VMEM((_CH, W), jnp.float32) for W in Ws]
    scratch += [pltpu.VMEM_SHARED((N, W), jnp.float32) for W in Ws]
    scratch += [pltpu.SemaphoreType.DMA]

    @functools.partial(pl.kernel, out_type=out_type, mesh=_sc_mesh(),
                       scratch_types=scratch)
    def k(*refs):
        nin = len(rows_list)
        r_hbm = refs[:nin]
        z_hbm = refs[nin:2 * nin]
        idx_hbm = refs[2 * nin]
        o_hbm = refs[2 * nin + 1:3 * nin + 1]
        idx_v = refs[3 * nin + 1]
        r_v = refs[3 * nin + 2:4 * nin + 2]
        acc = refs[4 * nin + 2:5 * nin + 2]
        sem = refs[5 * nin + 2]

        cid = lax.axis_index("c")
        sid = lax.axis_index("s")
        wid = sid * _NC + cid

        for a, z in zip(acc, z_hbm):
            pltpu.sync_copy(z.at[pl.ds(sid * rows_pt, rows_pt)],
                            a.at[pl.ds(sid * rows_pt, rows_pt)])
        plsc.subcore_barrier()

        def body(j, carry):
            c = wid + j * _NW

            @pl.when(c < chunks)
            def _():
                pltpu.sync_copy(idx_hbm.at[pl.ds(c * (_CH // _IB), _CH // _IB)],
                                idx_v)
                for rh, rv in zip(r_hbm, r_v):
                    pltpu.sync_copy(rh.at[pl.ds(c * _CH, _CH)], rv)
                for i in range(_CH // _IB):
                    for rv, a in zip(r_v, acc):
                        pltpu.sync_copy(rv.at[pl.ds(i * _IB, _IB)],
                                        a.at[idx_v.at[i]], add=True)

            return carry

        lax.fori_loop(0, nk, body, 0)
        plsc.subcore_barrier()
        for a, o in zip(acc, o_hbm):
            pltpu.sync_copy(a.at[pl.ds(sid * rows_pt, rows_pt)],
                            o.at[cid, pl.ds(sid * rows_pt, rows_pt)])

    zeros = [jnp.zeros((N, W), jnp.float32) for W in Ws]
    return k(*rows_list, *zeros, idx2d)


def _pad_rows(x, m=_CH):
    n = x.shape[0]
    p = (-n) % m
    return x if p == 0 else jnp.pad(x, ((0, p),) + ((0, 0),) * (x.ndim - 1))


def _pad_flat(idx, fill, m=_CH):
    n = idx.shape[0]
    p = (-n) % m
    if p:
        idx = jnp.concatenate([idx, jnp.full((p,), fill, jnp.int32)])
    return idx


def _blk(n, target):
    if n <= target:
        return n
    k = -(-n // target)
    while n % k:
        k += 1
    return n // k


def _gather_rows(tab, idx):
    return jnp.take(tab, idx, axis=0)


def _segsum(x, seg, n):
    return jax.ops.segment_sum(x, seg, num_segments=n)


def _softmax_stage(t, a8, seg, nseg):
    suma = _segsum(a8, seg, nseg)
    cnt = jnp.maximum(suma[:, 2:3], 1.0)
    amean = suma[:, 0:2] / cnt
    ex = jnp.exp(a8[:, 0:2] - amean[seg])
    den = _segsum(jnp.pad(ex, ((0, 0), (0, 6))), seg, nseg)
    exw = jnp.concatenate([jnp.repeat(ex[:, 0:1], D, 1),
                           jnp.repeat(ex[:, 1:2], D, 1)], axis=1)
    num = _segsum(exw * t, seg, nseg)
    return num, den


def _narrow_softmax(a8, seg, npad):
    suma = _segsum(a8, seg, npad)
    cnt = jnp.maximum(suma[:, 2:3], 1.0)
    amean8 = jnp.concatenate([suma[:, 0:2] / cnt,
                              jnp.zeros((npad, 6), jnp.float32)], axis=1)
    return amean8


def _atom_stage(pk, ef, src2d, dst, dst2d, p12, q12, h12, npad, molpk, eblk,
                nblk):
    gp = _sc_gather(p12, src2d, 2 * D)
    gq = _gather_rows(q12, dst)
    t, a8 = _edge_call(ef, gp, gq, pk['We'], pk['be'], pk['W2cat'],
                       pk['wa2pad'], pk['Watblk'], pk['batcat'], eblk)
    amean8 = _narrow_softmax(a8, dst, npad)
    gam = _gather_rows(amean8, dst)
    ext, ex8 = _scale_call(t, a8, gam, eblk)
    (num,) = _sc_scatter([ext], dst2d, npad)
    den = _segsum(ex8, dst, npad)
    den2 = jnp.stack([den, jnp.zeros_like(den)])
    return _atomctx_call(num, den2, h12, pk['gru'], molpk['wm2pad'],
                         molpk['rmb'], molpk['Watblk'], molpk['batcat'], nblk)


def _mol_stage(molpk, h12, rm12, tm12, n2g, n2g2d, nsegpad, nblk, sblk):
    (sparts,) = _sc_scatter([h12], n2g2d, nsegpad)
    s12, qm8 = _molqm_call(sparts, molpk['wm1blk'], molpk['qb'], sblk)
    gqm = _gather_rows(qm8, n2g)
    a8 = _a8_call(gqm, rm12, nblk)
    amean8 = _narrow_softmax(a8, n2g, nsegpad)
    gam = _gather_rows(amean8, n2g)
    ext, ex8 = _scale_call(tm12, a8, gam, nblk)
    (num,) = _sc_scatter([ext], n2g2d, nsegpad)
    den = _segsum(ex8, n2g, nsegpad)
    den2 = jnp.stack([den, jnp.zeros_like(den)])
    return _molctx_call(num, den2, s12, molpk['gru'], sblk)


def kernel(params, origin_node, origin_edge, origin_edge_index, frag_node,
           frag_edge, frag_edge_index, frag_node2graph, motif_node,
           motif_edge, motif_edge_index, motif_node2graph):
    V = frag_node.shape[0]
    E = frag_edge.shape[0]
    NF = motif_node.shape[0]
    EM = motif_edge.shape[0]
    G = NUM_GRAPHS
    Vp = V + ((-V) % _CH)
    EMp = EM + ((-EM) % _CH)
    NFp = NF + ((-NF) % _CH)
    Gp = G + ((-G) % _CH)

    fsrc = _pad_flat(jnp.asarray(frag_edge_index[0], jnp.int32), 0)
    fdst = _pad_flat(jnp.asarray(frag_edge_index[1], jnp.int32), Vp - 1)
    msrc = _pad_flat(jnp.asarray(motif_edge_index[0], jnp.int32), 0)
    mdst = _pad_flat(jnp.asarray(motif_edge_index[1], jnp.int32), NFp - 1)
    fn2g = _pad_flat(jnp.asarray(frag_node2graph, jnp.int32), NFp - 1)
    mn2g = _pad_flat(jnp.asarray(motif_node2graph, jnp.int32), Gp - 1)
    fsrc2d = fsrc.reshape(-1, _IB)
    fdst2d = fdst.reshape(-1, _IB)
    msrc2d = msrc.reshape(-1, _IB)
    mdst2d = mdst.reshape(-1, _IB)
    fn2g2d = fn2g.reshape(-1, _IB)
    mn2g2d = mn2g.reshape(-1, _IB)

    fh = params['frag_heads']
    jh = params['junc_heads']

    fa = _pack_atom([h['atom'] for h in fh])
    fa['We'] = params['emb_fe']['W']
    fa['be'] = params['emb_fe']['b'][None, :]
    fm = _pack_mol([h['mol'] for h in fh])
    ja = _pack_atom([h['atom'] for h in jh])
    ja['We'] = params['emb_me']['W']
    ja['be'] = params['emb_me']['b'][None, :]
    jm = _pack_mol([h['mol'] for h in jh])

    fn, p12, q12 = _pre_call(
        frag_node, params['emb_fn']['W'], params['emb_fn']['b'][None, :],
        fa['W1cat'], fa['b1cat'], fa['wa1cat'], fa['qb'], _blk(V, 2000))
    h12 = _pad_rows(jnp.concatenate([fn, fn], axis=1))
    hn12, rm12, tm12 = _atom_stage(fa, _pad_rows(frag_edge), fsrc2d, fdst,
                                   fdst2d, p12, q12, h12, Vp, fm,
                                   _blk(fsrc.shape[0], 2048), _blk(Vp, 2048))
    sf12 = _mol_stage(fm, hn12, rm12, tm12, fn2g, fn2g2d, NFp, _blk(Vp, 2048),
                      _blk(NFp, 2560))
    gm = _gm_call(sf12, params['frag_attend']['W'],
                  params['frag_attend']['b'][None, :], _blk(NFp, 2560))

    Pcat = jnp.concatenate([h['proj']['W'] for h in jh], axis=1)
    pbcat = jnp.concatenate([h['proj']['b'] for h in jh])[None, :]
    W1blk = block_diag(*[h['atom']['ne']['W'][:D] for h in jh])
    b1cat = jnp.concatenate([h['atom']['ne']['b'] for h in jh])[None, :]
    wa1blk = jnp.pad(block_diag(*[h['atom']['al']['W'][:D] for h in jh]),
                     ((0, 0), (0, 6)))
    jqb = jnp.array([jh[0]['atom']['al']['b'][0], jh[1]['atom']['al']['b'][0],
                     1.0, 0, 0, 0, 0, 0], jnp.float32)[None, :]
    x12, pj12, qj12 = _motifpre_call(motif_node, gm[:NF], params['emb_mn']['W'],
                                     params['emb_mn']['b'][None, :], Pcat,
                                     pbcat, W1blk, b1cat, wa1blk, jqb,
                                     _blk(NF, 2500))

    pj12 = _pad_rows(pj12)
    qj12 = _pad_rows(qj12)
    xn12, rj12, tj12 = _atom_stage(ja, _pad_rows(motif_edge), msrc2d, mdst,
                                   mdst2d, pj12, qj12, _pad_rows(x12), NFp, jm,
                                   _blk(EMp, 2048), _blk(NFp, 2560))
    sg12 = _mol_stage(jm, xn12, rj12, tj12, mn2g, mn2g2d, Gp, _blk(NFp, 2560),
                      Gp)

    out = _final_call(sg12, params['pred1']['W'],
                      params['pred1']['b'][None, :], params['pred2']['W'],
                      params['pred2']['b'][None, :])
    return out[:G]


# speedup vs baseline: 9.5691x; 3.4233x over previous
import functools

import jax
import jax.numpy as jnp
from jax import lax
from jax.experimental import pallas as pl
from jax.experimental.pallas import tpu as pltpu
from jax.experimental.pallas import tpu_sc as plsc
from jax.scipy.linalg import block_diag

D = 64
NEG = 0.01
NUM_GRAPHS = 250


def _lrelu(x):
    return jnp.where(x >= 0, x, NEG * x)


def _elu(x):
    return jnp.where(x > 0, x, jnp.exp(jnp.minimum(x, 0.0)) - 1.0)


def _sigmoid(x):
    return jax.nn.sigmoid(x)


def _pre_body(node_ref, We_ref, be_ref, W1_ref, b1_ref, wa1_ref, qb_ref,
              fn_ref, p_ref, q_ref):
    fn = _lrelu(jnp.dot(node_ref[...], We_ref[...],
                        preferred_element_type=jnp.float32) + be_ref[...])
    fn_ref[...] = fn
    p_ref[...] = jnp.dot(fn, W1_ref[...],
                         preferred_element_type=jnp.float32) + b1_ref[...]
    q_ref[...] = jnp.dot(fn, wa1_ref[...],
                         preferred_element_type=jnp.float32) + qb_ref[...]


def _pre_call(node, We, be, W1cat, b1cat, wa1cat, qbias, blk):
    n, din = node.shape
    grid = (n // blk,)
    return pl.pallas_call(
        _pre_body,
        grid=grid,
        in_specs=[
            pl.BlockSpec((blk, din), lambda i: (i, 0)),
            pl.BlockSpec(We.shape, lambda i: (0, 0)),
            pl.BlockSpec(be.shape, lambda i: (0, 0)),
            pl.BlockSpec(W1cat.shape, lambda i: (0, 0)),
            pl.BlockSpec(b1cat.shape, lambda i: (0, 0)),
            pl.BlockSpec(wa1cat.shape, lambda i: (0, 0)),
            pl.BlockSpec(qbias.shape, lambda i: (0, 0)),
        ],
        out_specs=[
            pl.BlockSpec((blk, D), lambda i: (i, 0)),
            pl.BlockSpec((blk, 2 * D), lambda i: (i, 0)),
            pl.BlockSpec((blk, 8), lambda i: (i, 0)),
        ],
        out_shape=[
            jax.ShapeDtypeStruct((n, D), jnp.float32),
            jax.ShapeDtypeStruct((n, 2 * D), jnp.float32),
            jax.ShapeDtypeStruct((n, 8), jnp.float32),
        ],
    )(node, We, be, W1cat, b1cat, wa1cat, qbias)


def _edge_body(ef_ref, gp_ref, We_ref, be_ref, W2_ref, wa2T_ref,
               Wat_ref, bat_ref, t_ref, rT_ref):
    fe = _lrelu(jnp.dot(ef_ref[...], We_ref[...],
                        preferred_element_type=jnp.float32) + be_ref[...])
    m = _lrelu(gp_ref[...] + jnp.dot(fe, W2_ref[...],
                                     preferred_element_type=jnp.float32))
    t_ref[...] = jnp.dot(m, Wat_ref[...],
                         preferred_element_type=jnp.float32) + bat_ref[...]
    rT_ref[...] = lax.dot_general(wa2T_ref[...], m, (((1,), (1,)), ((), ())),
                                  preferred_element_type=jnp.float32)


def _edge_call(ef, gp, We, be, W2cat, wa2T, Watblk, batcat, blk):
    e, dfe = ef.shape
    grid = (e // blk,)
    return pl.pallas_call(
        _edge_body,
        grid=grid,
        in_specs=[
            pl.BlockSpec((blk, dfe), lambda i: (i, 0)),
            pl.BlockSpec((blk, 2 * D), lambda i: (i, 0)),
            pl.BlockSpec(We.shape, lambda i: (0, 0)),
            pl.BlockSpec(be.shape, lambda i: (0, 0)),
            pl.BlockSpec(W2cat.shape, lambda i: (0, 0)),
            pl.BlockSpec(wa2T.shape, lambda i: (0, 0)),
            pl.BlockSpec(Watblk.shape, lambda i: (0, 0)),
            pl.BlockSpec(batcat.shape, lambda i: (0, 0)),
        ],
        out_specs=[
            pl.BlockSpec((blk, 2 * D), lambda i: (i, 0)),
            pl.BlockSpec((8, blk), lambda i: (0, i)),
        ],
        out_shape=[
            jax.ShapeDtypeStruct((e, 2 * D), jnp.float32),
            jax.ShapeDtypeStruct((8, e), jnp.float32),
        ],
    )(ef, gp, We, be, W2cat, wa2T, Watblk, batcat)


def _atomctx_body(num_ref, d1_ref, d2_ref, ones_ref, h_ref, Wz_ref, Uz_ref,
                  bz_ref, Wr_ref, Ur_ref, br_ref, Wn_ref, Un_ref, bn_ref,
                  wm2T_ref, Watm_ref, batm_ref, hn_ref, rmT_ref, tm_ref):
    num = num_ref[0] + num_ref[1]
    ones = ones_ref[...]
    dg = lambda p: lax.dot_general(p[...], ones, (((0,), (0,)), ((), ())),
                                   preferred_element_type=jnp.float32)
    den1 = dg(d1_ref) + 1e-16
    den2 = dg(d2_ref) + 1e-16
    lane = jax.lax.broadcasted_iota(jnp.int32, num.shape, 1)
    den = jnp.where(lane < D, den1, den2)
    ctx = _elu(num / den)
    h = h_ref[...]
    dot = lambda x, w: jnp.dot(x, w[...], preferred_element_type=jnp.float32)
    z = _sigmoid(dot(ctx, Wz_ref) + dot(h, Uz_ref) + bz_ref[...])
    r = _sigmoid(dot(ctx, Wr_ref) + dot(h, Ur_ref) + br_ref[...])
    n = jnp.tanh(dot(ctx, Wn_ref) + dot(r * h, Un_ref) + bn_ref[...])
    hn = jnp.maximum((1.0 - z) * n + z * h, 0.0)
    hn_ref[...] = hn
    rmT_ref[...] = lax.dot_general(wm2T_ref[...], hn, (((1,), (1,)), ((), ())),
                                   preferred_element_type=jnp.float32)
    tm_ref[...] = dot(hn, Watm_ref) + batm_ref[...]


def _atomctx_call(num, den, h12, gru, wm2T, Watmblk, batmcat, blk):
    n = num.shape[1]
    grid = (n // blk,)
    ones = jnp.ones((_NW, 1), jnp.float32)
    full = lambda a: pl.BlockSpec(a.shape, lambda i: (0, 0))
    return pl.pallas_call(
        _atomctx_body,
        grid=grid,
        in_specs=[
            pl.BlockSpec((2, blk, 2 * D), lambda i: (0, i, 0)),
            pl.BlockSpec((_NW, blk), lambda i: (0, i)),
            pl.BlockSpec((_NW, blk), lambda i: (0, i)),
            pl.BlockSpec((_NW, 1), lambda i: (0, 0)),
            pl.BlockSpec((blk, 2 * D), lambda i: (i, 0)),
            full(gru['Wz']), full(gru['Uz']), full(gru['bz']),
            full(gru['Wr']), full(gru['Ur']), full(gru['br']),
            full(gru['Wn']), full(gru['Un']), full(gru['bn']),
            full(wm2T), full(Watmblk), full(batmcat),
        ],
        out_specs=[
            pl.BlockSpec((blk, 2 * D), lambda i: (i, 0)),
            pl.BlockSpec((8, blk), lambda i: (0, i)),
            pl.BlockSpec((blk, 2 * D), lambda i: (i, 0)),
        ],
        out_shape=[
            jax.ShapeDtypeStruct((n, 2 * D), jnp.float32),
            jax.ShapeDtypeStruct((8, n), jnp.float32),
            jax.ShapeDtypeStruct((n, 2 * D), jnp.float32),
        ],
    )(num, den[0], den[1], ones, h12, gru['Wz'], gru['Uz'], gru['bz'],
      gru['Wr'], gru['Ur'], gru['br'], gru['Wn'], gru['Un'], gru['bn'], wm2T,
      Watmblk, batmcat)


def _molctx_body(num_ref, d1_ref, d2_ref, ones_ref, s_ref, Wz_ref, Uz_ref,
                 bz_ref, Wr_ref, Ur_ref, br_ref, Wn_ref, Un_ref, bn_ref,
                 sn_ref):
    num = num_ref[0] + num_ref[1]
    ones = ones_ref[...]
    dg = lambda p: lax.dot_general(p[...], ones, (((0,), (0,)), ((), ())),
                                   preferred_element_type=jnp.float32)
    den1 = dg(d1_ref) + 1e-16
    den2 = dg(d2_ref) + 1e-16
    lane = jax.lax.broadcasted_iota(jnp.int32, num.shape, 1)
    den = jnp.where(lane < D, den1, den2)
    ctx = _elu(num / den)
    s = s_ref[...]
    dot = lambda x, w: jnp.dot(x, w[...], preferred_element_type=jnp.float32)
    z = _sigmoid(dot(ctx, Wz_ref) + dot(s, Uz_ref) + bz_ref[...])
    r = _sigmoid(dot(ctx, Wr_ref) + dot(s, Ur_ref) + br_ref[...])
    n = jnp.tanh(dot(ctx, Wn_ref) + dot(r * s, Un_ref) + bn_ref[...])
    sn_ref[...] = (1.0 - z) * n + z * s


def _molctx_call(num, den, s12, gru, blk):
    n = num.shape[1]
    grid = (n // blk,)
    ones = jnp.ones((_NW, 1), jnp.float32)
    full = lambda a: pl.BlockSpec(a.shape, lambda i: (0, 0))
    return pl.pallas_call(
        _molctx_body,
        grid=grid,
        in_specs=[
            pl.BlockSpec((2, blk, 2 * D), lambda i: (0, i, 0)),
            pl.BlockSpec((_NW, blk), lambda i: (0, i)),
            pl.BlockSpec((_NW, blk), lambda i: (0, i)),
            pl.BlockSpec((_NW, 1), lambda i: (0, 0)),
            pl.BlockSpec((blk, 2 * D), lambda i: (i, 0)),
            full(gru['Wz']), full(gru['Uz']), full(gru['bz']),
            full(gru['Wr']), full(gru['Ur']), full(gru['br']),
            full(gru['Wn']), full(gru['Un']), full(gru['bn']),
        ],
        out_specs=pl.BlockSpec((blk, 2 * D), lambda i: (i, 0)),
        out_shape=jax.ShapeDtypeStruct((n, 2 * D), jnp.float32),
    )(num, den[0], den[1], ones, s12, gru['Wz'], gru['Uz'], gru['bz'],
      gru['Wr'], gru['Ur'], gru['br'], gru['Wn'], gru['Un'], gru['bn'])


def _motifpre_body(mnd_ref, gm_ref, Wemb_ref, bemb_ref, P_ref, pb_ref,
                   W1_ref, b1_ref, wa1_ref, qb_ref, x_ref, p_ref, q_ref):
    dot = lambda x, w: jnp.dot(x, w[...], preferred_element_type=jnp.float32)
    mn = _lrelu(dot(mnd_ref[...], Wemb_ref) + bemb_ref[...])
    mnode = jnp.concatenate([gm_ref[...], mn], axis=1)
    x = dot(mnode, P_ref) + pb_ref[...]
    x_ref[...] = x
    p_ref[...] = dot(x, W1_ref) + b1_ref[...]
    q_ref[...] = dot(x, wa1_ref) + qb_ref[...]


def _motifpre_call(motif_node, gm, Wemb, bemb, Pcat, pbcat, W1blk, b1cat,
                   wa1blk, qbias, blk):
    n, din = motif_node.shape
    grid = (n // blk,)
    full = lambda a: pl.BlockSpec(a.shape, lambda i: (0, 0))
    return pl.pallas_call(
        _motifpre_body,
        grid=grid,
        in_specs=[
            pl.BlockSpec((blk, din), lambda i: (i, 0)),
            pl.BlockSpec((blk, D), lambda i: (i, 0)),
            full(Wemb), full(bemb), full(Pcat), full(pbcat),
            full(W1blk), full(b1cat), full(wa1blk), full(qbias),
        ],
        out_specs=[
            pl.BlockSpec((blk, 2 * D), lambda i: (i, 0)),
            pl.BlockSpec((blk, 2 * D), lambda i: (i, 0)),
            pl.BlockSpec((blk, 8), lambda i: (i, 0)),
        ],
        out_shape=[
            jax.ShapeDtypeStruct((n, 2 * D), jnp.float32),
            jax.ShapeDtypeStruct((n, 2 * D), jnp.float32),
            jax.ShapeDtypeStruct((n, 8), jnp.float32),
        ],
    )(motif_node, gm, Wemb, bemb, Pcat, pbcat, W1blk, b1cat, wa1blk, qbias)


def _gm_body(sn_ref, Wfa_ref, bfa_ref, gm_ref):
    gm_ref[...] = jnp.maximum(
        jnp.dot(sn_ref[...], Wfa_ref[...],
                preferred_element_type=jnp.float32) + bfa_ref[...], 0.0)


def _gm_call(sn12, Wfa, bfa, blk):
    n = sn12.shape[0]
    grid = (n // blk,)
    return pl.pallas_call(
        _gm_body,
        grid=grid,
        in_specs=[
            pl.BlockSpec((blk, 2 * D), lambda i: (i, 0)),
            pl.BlockSpec(Wfa.shape, lambda i: (0, 0)),
            pl.BlockSpec(bfa.shape, lambda i: (0, 0)),
        ],
        out_specs=pl.BlockSpec((blk, D), lambda i: (i, 0)),
        out_shape=jax.ShapeDtypeStruct((n, D), jnp.float32),
    )(sn12, Wfa, bfa)


def _final_body(sn_ref, W1_ref, b1_ref, W2_ref, b2_ref, out_ref):
    sn = sn_ref[...]
    sup = jnp.maximum(0.5 * (sn[:, 0:D] + sn[:, D:2 * D]), 0.0)
    h1 = _lrelu(jnp.dot(sup, W1_ref[...],
                        preferred_element_type=jnp.float32) + b1_ref[...])
    out_ref[...] = jnp.dot(h1, W2_ref[...],
                           preferred_element_type=jnp.float32) + b2_ref[...]


def _final_call(sn12, W1, b1, W2, b2):
    n = sn12.shape[0]
    return pl.pallas_call(
        _final_body,
        in_specs=[pl.BlockSpec(sn12.shape, lambda: (0, 0)),
                  pl.BlockSpec(W1.shape, lambda: (0, 0)),
                  pl.BlockSpec(b1.shape, lambda: (0, 0)),
                  pl.BlockSpec(W2.shape, lambda: (0, 0)),
                  pl.BlockSpec(b2.shape, lambda: (0, 0))],
        out_specs=pl.BlockSpec((n, 1), lambda: (0, 0)),
        out_shape=jax.ShapeDtypeStruct((n, 1), jnp.float32),
    )(sn12, W1, b1, W2, b2)


def _pack_atom(heads):
    W1cat = jnp.concatenate([h['ne']['W'][:D] for h in heads], axis=1)
    b1cat = jnp.concatenate([h['ne']['b'] for h in heads])[None, :]
    W2cat = jnp.concatenate([h['ne']['W'][D:] for h in heads], axis=1)
    wa1 = jnp.concatenate([h['al']['W'][:D] for h in heads], axis=1)
    wa1cat = jnp.pad(wa1, ((0, 0), (0, 6)))
    qb = jnp.array([heads[0]['al']['b'][0], heads[1]['al']['b'][0],
                    1.0, 0, 0, 0, 0, 0], jnp.float32)[None, :]
    wa2 = block_diag(heads[0]['al']['W'][D:], heads[1]['al']['W'][D:])
    wa2pad = jnp.pad(wa2, ((0, 0), (0, 6)))
    Watblk = block_diag(heads[0]['at']['W'], heads[1]['at']['W'])
    batcat = jnp.concatenate([h['at']['b'] for h in heads])[None, :]
    gru = {k: block_diag(heads[0]['gru'][k], heads[1]['gru'][k])
           if heads[0]['gru'][k].ndim == 2
           else jnp.concatenate([h['gru'][k] for h in heads])[None, :]
           for k in heads[0]['gru']}
    return dict(W1cat=W1cat, b1cat=b1cat, W2cat=W2cat, wa1cat=wa1cat, qb=qb,
                wa2T=wa2pad.T, Watblk=Watblk, batcat=batcat, gru=gru)


def _pack_mol(heads):
    wm1blk = jnp.pad(block_diag(*[h['al']['W'][:D] for h in heads]),
                     ((0, 0), (0, 6)))
    qb = jnp.array([heads[0]['al']['b'][0], heads[1]['al']['b'][0],
                    1.0, 0, 0, 0, 0, 0], jnp.float32)[None, :]
    wm2 = block_diag(*[h['al']['W'][D:] for h in heads])
    wm2pad = jnp.pad(wm2, ((0, 0), (0, 6)))
    Watblk = block_diag(*[h['at']['W'] for h in heads])
    batcat = jnp.concatenate([h['at']['b'] for h in heads])[None, :]
    gru = {k: block_diag(*[h['gru'][k] for h in heads])
           if heads[0]['gru'][k].ndim == 2
           else jnp.concatenate([h['gru'][k] for h in heads])[None, :]
           for k in heads[0]['gru']}
    return dict(wm1blk=wm1blk, qb=qb, wm2T=wm2pad.T, Watblk=Watblk,
                batcat=batcat, gru=gru)


def _amean_body(p1_ref, p2_ref, p3_ref, ones_ref, am1_ref, am2_ref):
    ones = ones_ref[...]
    dg = lambda p: lax.dot_general(p[...], ones, (((0,), (0,)), ((), ())),
                                   preferred_element_type=jnp.float32)
    c = jnp.maximum(dg(p3_ref), 1.0)
    am1_ref[...] = dg(p1_ref) / c
    am2_ref[...] = dg(p2_ref) / c


def _amean_call(sparts, blk):
    n = sparts[0].shape[1]
    grid = (n // blk,)
    ones = jnp.ones((_NW, 1), jnp.float32)
    return pl.pallas_call(
        _amean_body,
        grid=grid,
        in_specs=[pl.BlockSpec((_NW, blk), lambda i: (0, i))] * 3 +
                 [pl.BlockSpec((_NW, 1), lambda i: (0, 0))],
        out_specs=[pl.BlockSpec((blk, 1), lambda i: (i, 0))] * 2,
        out_shape=[jax.ShapeDtypeStruct((n, 1), jnp.float32)] * 2,
    )(sparts[0], sparts[1], sparts[2], ones)


def _scale_body(t_ref, exT_ref, sel_ref, ext_ref):
    ex12 = lax.dot_general(exT_ref[...], sel_ref[...], (((0,), (0,)), ((), ())),
                           preferred_element_type=jnp.float32)
    t = t_ref[...]
    lane = jax.lax.broadcasted_iota(jnp.int32, t.shape, 1)
    ext_ref[...] = t * jnp.where(lane < D, ex12[:, 0:1], ex12[:, 1:2])


def _scale_call(t, exT, blk):
    n = t.shape[0]
    grid = (n // blk,)
    sel = jnp.zeros((8, 2), jnp.float32).at[0, 0].set(1.0).at[1, 1].set(1.0)
    return pl.pallas_call(
        _scale_body,
        grid=grid,
        in_specs=[
            pl.BlockSpec((blk, 2 * D), lambda i: (i, 0)),
            pl.BlockSpec((8, blk), lambda i: (0, i)),
            pl.BlockSpec((8, 2), lambda i: (0, 0)),
        ],
        out_specs=pl.BlockSpec((blk, 2 * D), lambda i: (i, 0)),
        out_shape=jax.ShapeDtypeStruct((n, 2 * D), jnp.float32),
    )(t, exT, sel)


def _a8_body(gqm_ref, rm_ref, out_ref):
    out_ref[...] = _lrelu(gqm_ref[...] + rm_ref[...])


def _a8_call(gqm, rm, blk):
    n = gqm.shape[0]
    grid = (n // blk,)
    return pl.pallas_call(
        _a8_body,
        grid=grid,
        in_specs=[pl.BlockSpec((blk, 8), lambda i: (i, 0)),
                  pl.BlockSpec((blk, 8), lambda i: (i, 0))],
        out_specs=pl.BlockSpec((blk, 8), lambda i: (i, 0)),
        out_shape=jax.ShapeDtypeStruct((n, 8), jnp.float32),
    )(gqm, rm)


def _molqm_body(sp_ref, wm1_ref, qb_ref, s_ref, qm_ref):
    s = sp_ref[0] + sp_ref[1]
    s_ref[...] = s
    qm_ref[...] = jnp.dot(s, wm1_ref[...],
                          preferred_element_type=jnp.float32) + qb_ref[...]


def _molqm_call(sparts, wm1blk, qb, blk):
    n = sparts.shape[1]
    grid = (n // blk,)
    return pl.pallas_call(
        _molqm_body,
        grid=grid,
        in_specs=[
            pl.BlockSpec((2, blk, 2 * D), lambda i: (0, i, 0)),
            pl.BlockSpec(wm1blk.shape, lambda i: (0, 0)),
            pl.BlockSpec(qb.shape, lambda i: (0, 0)),
        ],
        out_specs=[
            pl.BlockSpec((blk, 2 * D), lambda i: (i, 0)),
            pl.BlockSpec((blk, 8), lambda i: (i, 0)),
        ],
        out_shape=[
            jax.ShapeDtypeStruct((n, 2 * D), jnp.float32),
            jax.ShapeDtypeStruct((n, 8), jnp.float32),
        ],
    )(sparts, wm1blk, qb)


_NC, _NS, _NW = 2, 16, 32
_IB = 128
_CH = 256


def _sc_mesh():
    return plsc.VectorSubcoreMesh(core_axis_name="c", subcore_axis_name="s")


def _sc_gather(tab, idx2d, W):
    nrows = idx2d.shape[0] * _IB
    chunks = nrows // _CH
    nk = -(-chunks // _NW)

    @functools.partial(
        pl.kernel,
        out_type=jax.ShapeDtypeStruct((nrows, W), jnp.float32),
        mesh=_sc_mesh(),
        scratch_types=[
            pltpu.VMEM((_CH // _IB, _IB), jnp.int32),
            pltpu.VMEM((_CH, W), jnp.float32),
            pltpu.SemaphoreType.DMA,
        ],
    )
    def k(tab_hbm, idx_hbm, out_hbm, idx_v, rows_v, sem):
        wid = lax.axis_index("s") * _NC + lax.axis_index("c")

        def body(j, carry):
            c = wid + j * _NW

            @pl.when(c < chunks)
            def _():
                pltpu.sync_copy(idx_hbm.at[pl.ds(c * (_CH // _IB), _CH // _IB)],
                                idx_v)
                cps = [pltpu.async_copy(tab_hbm.at[idx_v.at[i]],
                                        rows_v.at[pl.ds(i * _IB, _IB)], sem)
                       for i in range(_CH // _IB)]
                for cp in cps:
                    cp.wait()
                pltpu.sync_copy(rows_v, out_hbm.at[pl.ds(c * _CH, _CH)])

            return carry

        lax.fori_loop(0, nk, body, 0)

    return k(tab, idx2d)


def _sc_scatter(rows_list, idx2d, N):
    nrows = idx2d.shape[0] * _IB
    chunks = nrows // _CH
    nk = -(-chunks // _NW)
    Ws = [r.shape[1] for r in rows_list]
    rows_pt = N // _NS

    out_type = [jax.ShapeDtypeStruct((_NC, N, W), jnp.float32) for W in Ws]
    scratch = [pltpu.VMEM((_CH // _IB, _IB), jnp.int32)]
    scratch += [pltpu.VMEM((_CH, W), jnp.float32) for W in Ws]
    scratch += [pltpu.VMEM_SHARED((N, W), jnp.float32) for W in Ws]
    scratch += [pltpu.SemaphoreType.DMA]

    @functools.partial(pl.kernel, out_type=out_type, mesh=_sc_mesh(),
                       scratch_types=scratch)
    def k(*refs):
        nin = len(rows_list)
        r_hbm = refs[:nin]
        z_hbm = refs[nin:2 * nin]
        idx_hbm = refs[2 * nin]
        o_hbm = refs[2 * nin + 1:3 * nin + 1]
        idx_v = refs[3 * nin + 1]
        r_v = refs[3 * nin + 2:4 * nin + 2]
        acc = refs[4 * nin + 2:5 * nin + 2]
        sem = refs[5 * nin + 2]

        cid = lax.axis_index("c")
        sid = lax.axis_index("s")
        wid = sid * _NC + cid

        for a, z in zip(acc, z_hbm):
            pltpu.sync_copy(z.at[pl.ds(sid * rows_pt, rows_pt)],
                            a.at[pl.ds(sid * rows_pt, rows_pt)])
        plsc.subcore_barrier()

        def body(j, carry):
            c = wid + j * _NW

            @pl.when(c < chunks)
            def _():
                pltpu.sync_copy(idx_hbm.at[pl.ds(c * (_CH // _IB), _CH // _IB)],
                                idx_v)
                for rh, rv in zip(r_hbm, r_v):
                    pltpu.sync_copy(rh.at[pl.ds(c * _CH, _CH)], rv)
                for i in range(_CH // _IB):
                    for rv, a in zip(r_v, acc):
                        pltpu.sync_copy(rv.at[pl.ds(i * _IB, _IB)],
                                        a.at[idx_v.at[i]], add=True)

            return carry

        lax.fori_loop(0, nk, body, 0)
        plsc.subcore_barrier()
        for a, o in zip(acc, o_hbm):
            pltpu.sync_copy(a.at[pl.ds(sid * rows_pt, rows_pt)],
                            o.at[cid, pl.ds(sid * rows_pt, rows_pt)])

    zeros = [jnp.zeros((N, W), jnp.float32) for W in Ws]
    return k(*rows_list, *zeros, idx2d)


def _sc_narrow(tab1, tab2, rT, idx2d, npad, mode):
    nrows = idx2d.shape[0] * _IB
    chunks = nrows // _CH
    nk = -(-chunks // _NW)
    T = tab1.shape[0]
    naccs = 3 if mode == 'stats' else 2
    zeros = jnp.zeros((npad,), jnp.float32)

    out_type = [jax.ShapeDtypeStruct((8, nrows), jnp.float32)]
    out_type += [jax.ShapeDtypeStruct((_NW, npad), jnp.float32)] * naccs
    scratch = [pltpu.VMEM((T,), jnp.float32), pltpu.VMEM((T,), jnp.float32),
               pltpu.VMEM((_CH // _IB, _IB), jnp.int32),
               pltpu.VMEM((8, _CH), jnp.float32),
               pltpu.VMEM((8, _CH), jnp.float32)]
    scratch += [pltpu.VMEM((npad,), jnp.float32)] * naccs
    scratch += [pltpu.SemaphoreType.DMA]

    @functools.partial(
        pl.kernel, out_type=out_type, mesh=_sc_mesh(),
        scratch_types=scratch,
        compiler_params=pltpu.CompilerParams(needs_layout_passes=False))
    def k(*refs):
        tab1_hbm, tab2_hbm, rT_hbm, idx_hbm, z_hbm = refs[:5]
        valT_hbm = refs[5]
        p_hbm = refs[6:6 + naccs]
        tab1_v, tab2_v, idx_v, r_v, v_v = refs[6 + naccs:11 + naccs]
        accs = refs[11 + naccs:11 + 2 * naccs]

        cid = lax.axis_index("c")
        sid = lax.axis_index("s")
        wid = sid * _NC + cid
        pltpu.sync_copy(tab1_hbm, tab1_v)
        pltpu.sync_copy(tab2_hbm, tab2_v)
        for a in accs:
            pltpu.sync_copy(z_hbm, a)
        one16 = jnp.ones((16,), jnp.float32)

        def body(j, carry):
            c = wid + j * _NW

            @pl.when(c < chunks)
            def _():
                pltpu.sync_copy(idx_hbm.at[pl.ds(c * (_CH // _IB),
                                                 _CH // _IB)], idx_v)
                pltpu.sync_copy(rT_hbm.at[:, pl.ds(c * _CH, _CH)], r_v)
                for i in range(_CH // _IB):
                    for g in range(_IB // 16):
                        off = i * _IB + g * 16
                        d = idx_v[i, pl.ds(g * 16, 16)]
                        t1 = plsc.load_gather(tab1_v, [d])
                        t2 = plsc.load_gather(tab2_v, [d])
                        r1 = r_v[0, pl.ds(off, 16)]
                        r2 = r_v[1, pl.ds(off, 16)]
                        if mode == 'stats':
                            v1 = _lrelu(t1 + r1)
                            v2 = _lrelu(t2 + r2)
                        else:
                            v1 = jnp.exp(r1 - t1)
                            v2 = jnp.exp(r2 - t2)
                        v_v[0, pl.ds(off, 16)] = v1
                        v_v[1, pl.ds(off, 16)] = v2
                        plsc.addupdate_scatter(accs[0], [d], v1)
                        plsc.addupdate_scatter(accs[1], [d], v2)
                        if mode == 'stats':
                            plsc.addupdate_scatter(accs[2], [d], one16)
                pltpu.sync_copy(v_v, valT_hbm.at[:, pl.ds(c * _CH, _CH)])

            return carry

        lax.fori_loop(0, nk, body, 0)
        for a, p in zip(accs, p_hbm):
            pltpu.sync_copy(a, p.at[wid])

    outs = k(tab1, tab2, rT, idx2d, zeros)
    return outs[0], outs[1:]


def _pad_rows(x, m=_CH):
    n = x.shape[0]
    p = (-n) % m
    return x if p == 0 else jnp.pad(x, ((0, p),) + ((0, 0),) * (x.ndim - 1))


def _pad_flat(idx, fill, m=_CH):
    n = idx.shape[0]
    p = (-n) % m
    if p:
        idx = jnp.concatenate([idx, jnp.full((p,), fill, jnp.int32)])
    return idx


def _blk(n, target):
    if n <= target:
        return n
    best = None
    b = 128
    while b <= target:
        if n % b == 0:
            best = b
        b += 128
    if best is None:
        k = -(-n // target)
        while n % k:
            k += 1
        best = n // k
    return best


def _gather_rows(tab, idx):
    return jnp.take(tab, idx, axis=0)


def _segsum(x, seg, n):
    return jax.ops.segment_sum(x, seg, num_segments=n)


def _softmax_stage(t, a8, seg, nseg):
    suma = _segsum(a8, seg, nseg)
    cnt = jnp.maximum(suma[:, 2:3], 1.0)
    amean = suma[:, 0:2] / cnt
    ex = jnp.exp(a8[:, 0:2] - amean[seg])
    den = _segsum(jnp.pad(ex, ((0, 0), (0, 6))), seg, nseg)
    exw = jnp.concatenate([jnp.repeat(ex[:, 0:1], D, 1),
                           jnp.repeat(ex[:, 1:2], D, 1)], axis=1)
    num = _segsum(exw * t, seg, nseg)
    return num, den


def _atom_stage(pk, ef, src2d, dst2d, p12, q2flat, h12, npad, molpk, eblk,
                nblk):
    gp = _sc_gather(p12, src2d, 2 * D)
    t, rT = _edge_call(ef, gp, pk['We'], pk['be'], pk['W2cat'], pk['wa2T'],
                       pk['Watblk'], pk['batcat'], eblk)
    aT, sparts = _sc_narrow(q2flat[0], q2flat[1], rT, dst2d, npad, 'stats')
    am1, am2 = _amean_call(sparts, _blk(npad, 2048))
    exT, den = _sc_narrow(am1.reshape(-1), am2.reshape(-1), aT, dst2d, npad,
                          'ex')
    ext = _scale_call(t, exT, eblk)
    (num,) = _sc_scatter([ext], dst2d, npad)
    return _atomctx_call(num, den, h12, pk['gru'], molpk['wm2T'],
                         molpk['Watblk'], molpk['batcat'], nblk)


def _mol_stage(molpk, h12, rmT, tm12, n2g2d, nsegpad, nblk, sblk):
    (sparts,) = _sc_scatter([h12], n2g2d, nsegpad)
    s12, qm8 = _molqm_call(sparts, molpk['wm1blk'], molpk['qb'], sblk)
    aT, stparts = _sc_narrow(qm8[:, 0], qm8[:, 1], rmT, n2g2d, nsegpad,
                             'stats')
    am1, am2 = _amean_call(stparts, sblk)
    exT, den = _sc_narrow(am1.reshape(-1), am2.reshape(-1), aT, n2g2d,
                          nsegpad, 'ex')
    ext = _scale_call(tm12, exT, nblk)
    (num,) = _sc_scatter([ext], n2g2d, nsegpad)
    return _molctx_call(num, den, s12, molpk['gru'], sblk)


def kernel(params, origin_node, origin_edge, origin_edge_index, frag_node,
           frag_edge, frag_edge_index, frag_node2graph, motif_node,
           motif_edge, motif_edge_index, motif_node2graph):
    V = frag_node.shape[0]
    E = frag_edge.shape[0]
    NF = motif_node.shape[0]
    EM = motif_edge.shape[0]
    G = NUM_GRAPHS
    Vp = V + ((-V) % _CH)
    EMp = EM + ((-EM) % _CH)
    NFp = NF + ((-NF) % _CH)
    Gp = G + ((-G) % _CH)

    fsrc = _pad_flat(jnp.asarray(frag_edge_index[0], jnp.int32), 0)
    fdst = _pad_flat(jnp.asarray(frag_edge_index[1], jnp.int32), Vp - 1)
    msrc = _pad_flat(jnp.asarray(motif_edge_index[0], jnp.int32), 0)
    mdst = _pad_flat(jnp.asarray(motif_edge_index[1], jnp.int32), NFp - 1)
    fn2g = _pad_flat(jnp.asarray(frag_node2graph, jnp.int32), NFp - 1)
    mn2g = _pad_flat(jnp.asarray(motif_node2graph, jnp.int32), Gp - 1)
    fsrc2d = fsrc.reshape(-1, _IB)
    fdst2d = fdst.reshape(-1, _IB)
    msrc2d = msrc.reshape(-1, _IB)
    mdst2d = mdst.reshape(-1, _IB)
    fn2g2d = fn2g.reshape(-1, _IB)
    mn2g2d = mn2g.reshape(-1, _IB)

    fh = params['frag_heads']
    jh = params['junc_heads']

    fa = _pack_atom([h['atom'] for h in fh])
    fa['We'] = params['emb_fe']['W']
    fa['be'] = params['emb_fe']['b'][None, :]
    fm = _pack_mol([h['mol'] for h in fh])
    ja = _pack_atom([h['atom'] for h in jh])
    ja['We'] = params['emb_me']['W']
    ja['be'] = params['emb_me']['b'][None, :]
    jm = _pack_mol([h['mol'] for h in jh])

    fn, p12, q12 = _pre_call(
        frag_node, params['emb_fn']['W'], params['emb_fn']['b'][None, :],
        fa['W1cat'], fa['b1cat'], fa['wa1cat'], fa['qb'], _blk(V, 2000))
    h12 = _pad_rows(jnp.concatenate([fn, fn], axis=1))
    q12p = _pad_rows(q12)
    q2flat = (q12p[:, 0], q12p[:, 1])
    hn12, rmT, tm12 = _atom_stage(fa, _pad_rows(frag_edge), fsrc2d,
                                  fdst2d, _pad_rows(p12), q2flat, h12, Vp, fm,
                                  _blk(fsrc.shape[0], 2048), _blk(Vp, 2048))
    sf12 = _mol_stage(fm, hn12, rmT, tm12, fn2g2d, NFp, _blk(Vp, 2048),
                      _blk(NFp, 2560))
    gm = _gm_call(sf12, params['frag_attend']['W'],
                  params['frag_attend']['b'][None, :], _blk(NFp, 2560))

    Pcat = jnp.concatenate([h['proj']['W'] for h in jh], axis=1)
    pbcat = jnp.concatenate([h['proj']['b'] for h in jh])[None, :]
    W1blk = block_diag(*[h['atom']['ne']['W'][:D] for h in jh])
    b1cat = jnp.concatenate([h['atom']['ne']['b'] for h in jh])[None, :]
    wa1blk = jnp.pad(block_diag(*[h['atom']['al']['W'][:D] for h in jh]),
                     ((0, 0), (0, 6)))
    jqb = jnp.array([jh[0]['atom']['al']['b'][0], jh[1]['atom']['al']['b'][0],
                     1.0, 0, 0, 0, 0, 0], jnp.float32)[None, :]
    x12, pj12, qj12 = _motifpre_call(motif_node, gm[:NF], params['emb_mn']['W'],
                                     params['emb_mn']['b'][None, :], Pcat,
                                     pbcat, W1blk, b1cat, wa1blk, jqb,
                                     _blk(NF, 2500))

    pj12 = _pad_rows(pj12)
    qj12p = _pad_rows(qj12)
    qj2flat = (qj12p[:, 0], qj12p[:, 1])
    xn12, rjT, tj12 = _atom_stage(ja, _pad_rows(motif_edge), msrc2d,
                                  mdst2d, pj12, qj2flat, _pad_rows(x12), NFp,
                                  jm, _blk(EMp, 2048), _blk(NFp, 2560))
    sg12 = _mol_stage(jm, xn12, rjT, tj12, mn2g2d, Gp, _blk(NFp, 2560), Gp)

    out = _final_call(sg12, params['pred1']['W'],
                      params['pred1']['b'][None, :], params['pred2']['W'],
                      params['pred2']['b'][None, :])
    return out[:G]


# speedup vs baseline: 9.9310x; 1.0378x over previous
import functools

import jax
import jax.numpy as jnp
from jax import lax
from jax.experimental import pallas as pl
from jax.experimental.pallas import tpu as pltpu
from jax.experimental.pallas import tpu_sc as plsc
from jax.scipy.linalg import block_diag

D = 64
NEG = 0.01
NUM_GRAPHS = 250


def _lrelu(x):
    return jnp.where(x >= 0, x, NEG * x)


def _elu(x):
    return jnp.where(x > 0, x, jnp.exp(jnp.minimum(x, 0.0)) - 1.0)


def _sigmoid(x):
    return jax.nn.sigmoid(x)


def _pre_body(node_ref, We_ref, be_ref, W1_ref, b1_ref, wa1_ref, qb_ref,
              fn_ref, p_ref, q_ref):
    fn = _lrelu(jnp.dot(node_ref[...], We_ref[...],
                        preferred_element_type=jnp.float32) + be_ref[...])
    fn_ref[...] = fn
    p_ref[...] = jnp.dot(fn, W1_ref[...],
                         preferred_element_type=jnp.float32) + b1_ref[...]
    q_ref[...] = jnp.dot(fn, wa1_ref[...],
                         preferred_element_type=jnp.float32) + qb_ref[...]


def _pre_call(node, We, be, W1cat, b1cat, wa1cat, qbias, blk):
    n, din = node.shape
    grid = (n // blk,)
    return pl.pallas_call(
        _pre_body,
        grid=grid,
        in_specs=[
            pl.BlockSpec((blk, din), lambda i: (i, 0)),
            pl.BlockSpec(We.shape, lambda i: (0, 0)),
            pl.BlockSpec(be.shape, lambda i: (0, 0)),
            pl.BlockSpec(W1cat.shape, lambda i: (0, 0)),
            pl.BlockSpec(b1cat.shape, lambda i: (0, 0)),
            pl.BlockSpec(wa1cat.shape, lambda i: (0, 0)),
            pl.BlockSpec(qbias.shape, lambda i: (0, 0)),
        ],
        out_specs=[
            pl.BlockSpec((blk, D), lambda i: (i, 0)),
            pl.BlockSpec((blk, 2 * D), lambda i: (i, 0)),
            pl.BlockSpec((blk, 8), lambda i: (i, 0)),
        ],
        out_shape=[
            jax.ShapeDtypeStruct((n, D), jnp.float32),
            jax.ShapeDtypeStruct((n, 2 * D), jnp.float32),
            jax.ShapeDtypeStruct((n, 8), jnp.float32),
        ],
    )(node, We, be, W1cat, b1cat, wa1cat, qbias)


def _edge_body(ef_ref, gp_ref, We_ref, be_ref, W2_ref, wa2T_ref,
               Wat_ref, bat_ref, t_ref, rT_ref):
    fe = _lrelu(jnp.dot(ef_ref[...], We_ref[...],
                        preferred_element_type=jnp.float32) + be_ref[...])
    m = _lrelu(gp_ref[...] + jnp.dot(fe, W2_ref[...],
                                     preferred_element_type=jnp.float32))
    t_ref[...] = jnp.dot(m, Wat_ref[...],
                         preferred_element_type=jnp.float32) + bat_ref[...]
    rT_ref[...] = lax.dot_general(wa2T_ref[...], m, (((1,), (1,)), ((), ())),
                                  preferred_element_type=jnp.float32)


def _edge_call(ef, gp, We, be, W2cat, wa2T, Watblk, batcat, blk):
    e, dfe = ef.shape
    grid = (e // blk,)
    return pl.pallas_call(
        _edge_body,
        grid=grid,
        in_specs=[
            pl.BlockSpec((blk, dfe), lambda i: (i, 0)),
            pl.BlockSpec((blk, 2 * D), lambda i: (i, 0)),
            pl.BlockSpec(We.shape, lambda i: (0, 0)),
            pl.BlockSpec(be.shape, lambda i: (0, 0)),
            pl.BlockSpec(W2cat.shape, lambda i: (0, 0)),
            pl.BlockSpec(wa2T.shape, lambda i: (0, 0)),
            pl.BlockSpec(Watblk.shape, lambda i: (0, 0)),
            pl.BlockSpec(batcat.shape, lambda i: (0, 0)),
        ],
        out_specs=[
            pl.BlockSpec((blk, 2 * D), lambda i: (i, 0)),
            pl.BlockSpec((8, blk), lambda i: (0, i)),
        ],
        out_shape=[
            jax.ShapeDtypeStruct((e, 2 * D), jnp.float32),
            jax.ShapeDtypeStruct((8, e), jnp.float32),
        ],
    )(ef, gp, We, be, W2cat, wa2T, Watblk, batcat)


def _atomctx_body(num_ref, d1_ref, d2_ref, ones_ref, h_ref, Wz_ref, Uz_ref,
                  bz_ref, Wr_ref, Ur_ref, br_ref, Wn_ref, Un_ref, bn_ref,
                  wm2T_ref, Watm_ref, batm_ref, hn_ref, rmT_ref, tm_ref):
    num = num_ref[0] + num_ref[1]
    ones = ones_ref[...]
    dg = lambda p: lax.dot_general(p[...], ones, (((0,), (0,)), ((), ())),
                                   preferred_element_type=jnp.float32)
    den1 = dg(d1_ref) + 1e-16
    den2 = dg(d2_ref) + 1e-16
    lane = jax.lax.broadcasted_iota(jnp.int32, num.shape, 1)
    den = jnp.where(lane < D, den1, den2)
    ctx = _elu(num / den)
    h = h_ref[...]
    dot = lambda x, w: jnp.dot(x, w[...], preferred_element_type=jnp.float32)
    z = _sigmoid(dot(ctx, Wz_ref) + dot(h, Uz_ref) + bz_ref[...])
    r = _sigmoid(dot(ctx, Wr_ref) + dot(h, Ur_ref) + br_ref[...])
    n = jnp.tanh(dot(ctx, Wn_ref) + dot(r * h, Un_ref) + bn_ref[...])
    hn = jnp.maximum((1.0 - z) * n + z * h, 0.0)
    hn_ref[...] = hn
    rmT_ref[...] = lax.dot_general(wm2T_ref[...], hn, (((1,), (1,)), ((), ())),
                                   preferred_element_type=jnp.float32)
    tm_ref[...] = dot(hn, Watm_ref) + batm_ref[...]


def _atomctx_call(num, den, h12, gru, wm2T, Watmblk, batmcat, blk):
    n = num.shape[1]
    grid = (n // blk,)
    ones = jnp.ones((_NW, 1), jnp.float32)
    full = lambda a: pl.BlockSpec(a.shape, lambda i: (0, 0))
    return pl.pallas_call(
        _atomctx_body,
        grid=grid,
        in_specs=[
            pl.BlockSpec((2, blk, 2 * D), lambda i: (0, i, 0)),
            pl.BlockSpec((_NW, blk), lambda i: (0, i)),
            pl.BlockSpec((_NW, blk), lambda i: (0, i)),
            pl.BlockSpec((_NW, 1), lambda i: (0, 0)),
            pl.BlockSpec((blk, 2 * D), lambda i: (i, 0)),
            full(gru['Wz']), full(gru['Uz']), full(gru['bz']),
            full(gru['Wr']), full(gru['Ur']), full(gru['br']),
            full(gru['Wn']), full(gru['Un']), full(gru['bn']),
            full(wm2T), full(Watmblk), full(batmcat),
        ],
        out_specs=[
            pl.BlockSpec((blk, 2 * D), lambda i: (i, 0)),
            pl.BlockSpec((8, blk), lambda i: (0, i)),
            pl.BlockSpec((blk, 2 * D), lambda i: (i, 0)),
        ],
        out_shape=[
            jax.ShapeDtypeStruct((n, 2 * D), jnp.float32),
            jax.ShapeDtypeStruct((8, n), jnp.float32),
            jax.ShapeDtypeStruct((n, 2 * D), jnp.float32),
        ],
    )(num, den[0], den[1], ones, h12, gru['Wz'], gru['Uz'], gru['bz'],
      gru['Wr'], gru['Ur'], gru['br'], gru['Wn'], gru['Un'], gru['bn'], wm2T,
      Watmblk, batmcat)


def _molctx_body(num_ref, d1_ref, d2_ref, ones_ref, s_ref, Wz_ref, Uz_ref,
                 bz_ref, Wr_ref, Ur_ref, br_ref, Wn_ref, Un_ref, bn_ref,
                 sn_ref):
    num = num_ref[0] + num_ref[1]
    ones = ones_ref[...]
    dg = lambda p: lax.dot_general(p[...], ones, (((0,), (0,)), ((), ())),
                                   preferred_element_type=jnp.float32)
    den1 = dg(d1_ref) + 1e-16
    den2 = dg(d2_ref) + 1e-16
    lane = jax.lax.broadcasted_iota(jnp.int32, num.shape, 1)
    den = jnp.where(lane < D, den1, den2)
    ctx = _elu(num / den)
    s = s_ref[...]
    dot = lambda x, w: jnp.dot(x, w[...], preferred_element_type=jnp.float32)
    z = _sigmoid(dot(ctx, Wz_ref) + dot(s, Uz_ref) + bz_ref[...])
    r = _sigmoid(dot(ctx, Wr_ref) + dot(s, Ur_ref) + br_ref[...])
    n = jnp.tanh(dot(ctx, Wn_ref) + dot(r * s, Un_ref) + bn_ref[...])
    sn_ref[...] = (1.0 - z) * n + z * s


def _molctx_call(num, den, s12, gru, blk):
    n = num.shape[1]
    grid = (n // blk,)
    ones = jnp.ones((_NW, 1), jnp.float32)
    full = lambda a: pl.BlockSpec(a.shape, lambda i: (0, 0))
    return pl.pallas_call(
        _molctx_body,
        grid=grid,
        in_specs=[
            pl.BlockSpec((2, blk, 2 * D), lambda i: (0, i, 0)),
            pl.BlockSpec((_NW, blk), lambda i: (0, i)),
            pl.BlockSpec((_NW, blk), lambda i: (0, i)),
            pl.BlockSpec((_NW, 1), lambda i: (0, 0)),
            pl.BlockSpec((blk, 2 * D), lambda i: (i, 0)),
            full(gru['Wz']), full(gru['Uz']), full(gru['bz']),
            full(gru['Wr']), full(gru['Ur']), full(gru['br']),
            full(gru['Wn']), full(gru['Un']), full(gru['bn']),
        ],
        out_specs=pl.BlockSpec((blk, 2 * D), lambda i: (i, 0)),
        out_shape=jax.ShapeDtypeStruct((n, 2 * D), jnp.float32),
    )(num, den[0], den[1], ones, s12, gru['Wz'], gru['Uz'], gru['bz'],
      gru['Wr'], gru['Ur'], gru['br'], gru['Wn'], gru['Un'], gru['bn'])


def _motifpre_body(mnd_ref, gm_ref, Wemb_ref, bemb_ref, P_ref, pb_ref,
                   W1_ref, b1_ref, wa1_ref, qb_ref, x_ref, p_ref, q_ref):
    dot = lambda x, w: jnp.dot(x, w[...], preferred_element_type=jnp.float32)
    mn = _lrelu(dot(mnd_ref[...], Wemb_ref) + bemb_ref[...])
    mnode = jnp.concatenate([gm_ref[...], mn], axis=1)
    x = dot(mnode, P_ref) + pb_ref[...]
    x_ref[...] = x
    p_ref[...] = dot(x, W1_ref) + b1_ref[...]
    q_ref[...] = dot(x, wa1_ref) + qb_ref[...]


def _motifpre_call(motif_node, gm, Wemb, bemb, Pcat, pbcat, W1blk, b1cat,
                   wa1blk, qbias, blk):
    n, din = motif_node.shape
    grid = (n // blk,)
    full = lambda a: pl.BlockSpec(a.shape, lambda i: (0, 0))
    return pl.pallas_call(
        _motifpre_body,
        grid=grid,
        in_specs=[
            pl.BlockSpec((blk, din), lambda i: (i, 0)),
            pl.BlockSpec((blk, D), lambda i: (i, 0)),
            full(Wemb), full(bemb), full(Pcat), full(pbcat),
            full(W1blk), full(b1cat), full(wa1blk), full(qbias),
        ],
        out_specs=[
            pl.BlockSpec((blk, 2 * D), lambda i: (i, 0)),
            pl.BlockSpec((blk, 2 * D), lambda i: (i, 0)),
            pl.BlockSpec((blk, 8), lambda i: (i, 0)),
        ],
        out_shape=[
            jax.ShapeDtypeStruct((n, 2 * D), jnp.float32),
            jax.ShapeDtypeStruct((n, 2 * D), jnp.float32),
            jax.ShapeDtypeStruct((n, 8), jnp.float32),
        ],
    )(motif_node, gm, Wemb, bemb, Pcat, pbcat, W1blk, b1cat, wa1blk, qbias)


def _gm_body(sn_ref, Wfa_ref, bfa_ref, gm_ref):
    gm_ref[...] = jnp.maximum(
        jnp.dot(sn_ref[...], Wfa_ref[...],
                preferred_element_type=jnp.float32) + bfa_ref[...], 0.0)


def _gm_call(sn12, Wfa, bfa, blk):
    n = sn12.shape[0]
    grid = (n // blk,)
    return pl.pallas_call(
        _gm_body,
        grid=grid,
        in_specs=[
            pl.BlockSpec((blk, 2 * D), lambda i: (i, 0)),
            pl.BlockSpec(Wfa.shape, lambda i: (0, 0)),
            pl.BlockSpec(bfa.shape, lambda i: (0, 0)),
        ],
        out_specs=pl.BlockSpec((blk, D), lambda i: (i, 0)),
        out_shape=jax.ShapeDtypeStruct((n, D), jnp.float32),
    )(sn12, Wfa, bfa)


def _final_body(sn_ref, W1_ref, b1_ref, W2_ref, b2_ref, out_ref):
    sn = sn_ref[...]
    sup = jnp.maximum(0.5 * (sn[:, 0:D] + sn[:, D:2 * D]), 0.0)
    h1 = _lrelu(jnp.dot(sup, W1_ref[...],
                        preferred_element_type=jnp.float32) + b1_ref[...])
    out_ref[...] = jnp.dot(h1, W2_ref[...],
                           preferred_element_type=jnp.float32) + b2_ref[...]


def _final_call(sn12, W1, b1, W2, b2):
    n = sn12.shape[0]
    return pl.pallas_call(
        _final_body,
        in_specs=[pl.BlockSpec(sn12.shape, lambda: (0, 0)),
                  pl.BlockSpec(W1.shape, lambda: (0, 0)),
                  pl.BlockSpec(b1.shape, lambda: (0, 0)),
                  pl.BlockSpec(W2.shape, lambda: (0, 0)),
                  pl.BlockSpec(b2.shape, lambda: (0, 0))],
        out_specs=pl.BlockSpec((n, 1), lambda: (0, 0)),
        out_shape=jax.ShapeDtypeStruct((n, 1), jnp.float32),
    )(sn12, W1, b1, W2, b2)


def _pack_atom(heads):
    W1cat = jnp.concatenate([h['ne']['W'][:D] for h in heads], axis=1)
    b1cat = jnp.concatenate([h['ne']['b'] for h in heads])[None, :]
    W2cat = jnp.concatenate([h['ne']['W'][D:] for h in heads], axis=1)
    wa1 = jnp.concatenate([h['al']['W'][:D] for h in heads], axis=1)
    wa1cat = jnp.pad(wa1, ((0, 0), (0, 6)))
    qb = jnp.array([heads[0]['al']['b'][0], heads[1]['al']['b'][0],
                    1.0, 0, 0, 0, 0, 0], jnp.float32)[None, :]
    wa2 = block_diag(heads[0]['al']['W'][D:], heads[1]['al']['W'][D:])
    wa2pad = jnp.pad(wa2, ((0, 0), (0, 6)))
    Watblk = block_diag(heads[0]['at']['W'], heads[1]['at']['W'])
    batcat = jnp.concatenate([h['at']['b'] for h in heads])[None, :]
    gru = {k: block_diag(heads[0]['gru'][k], heads[1]['gru'][k])
           if heads[0]['gru'][k].ndim == 2
           else jnp.concatenate([h['gru'][k] for h in heads])[None, :]
           for k in heads[0]['gru']}
    return dict(W1cat=W1cat, b1cat=b1cat, W2cat=W2cat, wa1cat=wa1cat, qb=qb,
                wa2T=wa2pad.T, Watblk=Watblk, batcat=batcat, gru=gru)


def _pack_mol(heads):
    wm1blk = jnp.pad(block_diag(*[h['al']['W'][:D] for h in heads]),
                     ((0, 0), (0, 6)))
    qb = jnp.array([heads[0]['al']['b'][0], heads[1]['al']['b'][0],
                    1.0, 0, 0, 0, 0, 0], jnp.float32)[None, :]
    wm2 = block_diag(*[h['al']['W'][D:] for h in heads])
    wm2pad = jnp.pad(wm2, ((0, 0), (0, 6)))
    Watblk = block_diag(*[h['at']['W'] for h in heads])
    batcat = jnp.concatenate([h['at']['b'] for h in heads])[None, :]
    gru = {k: block_diag(*[h['gru'][k] for h in heads])
           if heads[0]['gru'][k].ndim == 2
           else jnp.concatenate([h['gru'][k] for h in heads])[None, :]
           for k in heads[0]['gru']}
    return dict(wm1blk=wm1blk, qb=qb, wm2T=wm2pad.T, Watblk=Watblk,
                batcat=batcat, gru=gru)


def _amean_body(p1_ref, p2_ref, p3_ref, ones_ref, am1_ref, am2_ref):
    ones = ones_ref[...]
    dg = lambda p: lax.dot_general(p[...], ones, (((0,), (0,)), ((), ())),
                                   preferred_element_type=jnp.float32)
    c = jnp.maximum(dg(p3_ref), 1.0)
    am1_ref[...] = dg(p1_ref) / c
    am2_ref[...] = dg(p2_ref) / c


def _amean_call(sparts, blk):
    n = sparts[0].shape[1]
    grid = (n // blk,)
    ones = jnp.ones((_NW, 1), jnp.float32)
    return pl.pallas_call(
        _amean_body,
        grid=grid,
        in_specs=[pl.BlockSpec((_NW, blk), lambda i: (0, i))] * 3 +
                 [pl.BlockSpec((_NW, 1), lambda i: (0, 0))],
        out_specs=[pl.BlockSpec((blk, 1), lambda i: (i, 0))] * 2,
        out_shape=[jax.ShapeDtypeStruct((n, 1), jnp.float32)] * 2,
    )(sparts[0], sparts[1], sparts[2], ones)


def _scale_body(t_ref, exT_ref, sel_ref, ext_ref):
    ex12 = lax.dot_general(exT_ref[...], sel_ref[...], (((0,), (0,)), ((), ())),
                           preferred_element_type=jnp.float32)
    t = t_ref[...]
    lane = jax.lax.broadcasted_iota(jnp.int32, t.shape, 1)
    ext_ref[...] = t * jnp.where(lane < D, ex12[:, 0:1], ex12[:, 1:2])


def _scale_call(t, exT, blk):
    n = t.shape[0]
    grid = (n // blk,)
    sel = jnp.zeros((8, 2), jnp.float32).at[0, 0].set(1.0).at[1, 1].set(1.0)
    return pl.pallas_call(
        _scale_body,
        grid=grid,
        in_specs=[
            pl.BlockSpec((blk, 2 * D), lambda i: (i, 0)),
            pl.BlockSpec((8, blk), lambda i: (0, i)),
            pl.BlockSpec((8, 2), lambda i: (0, 0)),
        ],
        out_specs=pl.BlockSpec((blk, 2 * D), lambda i: (i, 0)),
        out_shape=jax.ShapeDtypeStruct((n, 2 * D), jnp.float32),
    )(t, exT, sel)


def _a8_body(gqm_ref, rm_ref, out_ref):
    out_ref[...] = _lrelu(gqm_ref[...] + rm_ref[...])


def _a8_call(gqm, rm, blk):
    n = gqm.shape[0]
    grid = (n // blk,)
    return pl.pallas_call(
        _a8_body,
        grid=grid,
        in_specs=[pl.BlockSpec((blk, 8), lambda i: (i, 0)),
                  pl.BlockSpec((blk, 8), lambda i: (i, 0))],
        out_specs=pl.BlockSpec((blk, 8), lambda i: (i, 0)),
        out_shape=jax.ShapeDtypeStruct((n, 8), jnp.float32),
    )(gqm, rm)


def _molqm_body(sp_ref, wm1_ref, qb_ref, s_ref, qm_ref):
    s = sp_ref[0] + sp_ref[1]
    s_ref[...] = s
    qm_ref[...] = jnp.dot(s, wm1_ref[...],
                          preferred_element_type=jnp.float32) + qb_ref[...]


def _molqm_call(sparts, wm1blk, qb, blk):
    n = sparts.shape[1]
    grid = (n // blk,)
    return pl.pallas_call(
        _molqm_body,
        grid=grid,
        in_specs=[
            pl.BlockSpec((2, blk, 2 * D), lambda i: (0, i, 0)),
            pl.BlockSpec(wm1blk.shape, lambda i: (0, 0)),
            pl.BlockSpec(qb.shape, lambda i: (0, 0)),
        ],
        out_specs=[
            pl.BlockSpec((blk, 2 * D), lambda i: (i, 0)),
            pl.BlockSpec((blk, 8), lambda i: (i, 0)),
        ],
        out_shape=[
            jax.ShapeDtypeStruct((n, 2 * D), jnp.float32),
            jax.ShapeDtypeStruct((n, 8), jnp.float32),
        ],
    )(sparts, wm1blk, qb)


_NC, _NS, _NW = 2, 16, 32
_IB = 128
_CH = 512


def _sc_mesh():
    return plsc.VectorSubcoreMesh(core_axis_name="c", subcore_axis_name="s")


def _ch_for(nrows, want):
    c = min(want, nrows)
    while nrows % c:
        c -= _IB
    return c


def _sc_gather(tab, idx3d, W):
    ch = idx3d.shape[1] * _IB
    chunks = idx3d.shape[0]
    nrows = chunks * ch
    nk = -(-chunks // _NW)

    @functools.partial(
        pl.kernel,
        out_type=jax.ShapeDtypeStruct((nrows, W), jnp.float32),
        mesh=_sc_mesh(),
        scratch_types=[
            pltpu.VMEM((ch // _IB, _IB), jnp.int32),
            pltpu.VMEM((ch, W), jnp.float32),
            pltpu.SemaphoreType.DMA,
        ],
    )
    def k(tab_hbm, idx_hbm, out_hbm, idx_v, rows_v, sem):
        wid = lax.axis_index("s") * _NC + lax.axis_index("c")

        def body(j, carry):
            c = wid + j * _NW

            @pl.when(c < chunks)
            def _():
                pltpu.sync_copy(idx_hbm.at[c], idx_v)
                cps = [pltpu.async_copy(tab_hbm.at[idx_v.at[i]],
                                        rows_v.at[pl.ds(i * _IB, _IB)], sem)
                       for i in range(ch // _IB)]
                for cp in cps:
                    cp.wait()
                pltpu.sync_copy(rows_v, out_hbm.at[pl.ds(c * ch, ch)])

            return carry

        lax.fori_loop(0, nk, body, 0)

    return k(tab, idx3d)


def _sc_scatter(rows_list, idx3d, N):
    ch = idx3d.shape[1] * _IB
    chunks = idx3d.shape[0]
    nrows = chunks * ch
    nk = -(-chunks // _NW)
    Ws = [r.shape[1] for r in rows_list]
    rows_pt = N // _NS

    out_type = [jax.ShapeDtypeStruct((_NC, N, W), jnp.float32) for W in Ws]
    scratch = [pltpu.VMEM((ch // _IB, _IB), jnp.int32)]
    scratch += [pltpu.VMEM((ch, W), jnp.float32) for W in Ws]
    scratch += [pltpu.VMEM_SHARED((N, W), jnp.float32) for W in Ws]
    scratch += [pltpu.SemaphoreType.DMA]

    @functools.partial(pl.kernel, out_type=out_type, mesh=_sc_mesh(),
                       scratch_types=scratch)
    def k(*refs):
        nin = len(rows_list)
        r_hbm = refs[:nin]
        z_hbm = refs[nin:2 * nin]
        idx_hbm = refs[2 * nin]
        o_hbm = refs[2 * nin + 1:3 * nin + 1]
        idx_v = refs[3 * nin + 1]
        r_v = refs[3 * nin + 2:4 * nin + 2]
        acc = refs[4 * nin + 2:5 * nin + 2]
        sem = refs[5 * nin + 2]

        cid = lax.axis_index("c")
        sid = lax.axis_index("s")
        wid = sid * _NC + cid

        for a, z in zip(acc, z_hbm):
            pltpu.sync_copy(z.at[pl.ds(sid * rows_pt, rows_pt)],
                            a.at[pl.ds(sid * rows_pt, rows_pt)])
        plsc.subcore_barrier()

        def body(j, carry):
            c = wid + j * _NW

            @pl.when(c < chunks)
            def _():
                pltpu.sync_copy(idx_hbm.at[c], idx_v)
                for rh, rv in zip(r_hbm, r_v):
                    pltpu.sync_copy(rh.at[pl.ds(c * ch, ch)], rv)
                for i in range(ch // _IB):
                    for rv, a in zip(r_v, acc):
                        pltpu.sync_copy(rv.at[pl.ds(i * _IB, _IB)],
                                        a.at[idx_v.at[i]], add=True)

            return carry

        lax.fori_loop(0, nk, body, 0)
        plsc.subcore_barrier()
        for a, o in zip(acc, o_hbm):
            pltpu.sync_copy(a.at[pl.ds(sid * rows_pt, rows_pt)],
                            o.at[cid, pl.ds(sid * rows_pt, rows_pt)])

    zeros = [jnp.zeros((N, W), jnp.float32) for W in Ws]
    return k(*rows_list, *zeros, idx3d)


def _sc_narrow(tab1, tab2, rT, idxflat, npad, mode):
    nrows = idxflat.shape[0]
    ch = _ch_for(nrows, 640)
    chunks = nrows // ch
    nk = -(-chunks // _NW)
    T = tab1.shape[0]
    naccs = 3 if mode == 'stats' else 2
    zeros = jnp.zeros((npad,), jnp.float32)

    out_type = [jax.ShapeDtypeStruct((8, nrows), jnp.float32)]
    out_type += [jax.ShapeDtypeStruct((_NW, npad), jnp.float32)] * naccs
    scratch = [pltpu.VMEM((T,), jnp.float32), pltpu.VMEM((T,), jnp.float32),
               pltpu.VMEM((ch,), jnp.int32),
               pltpu.VMEM((8, ch), jnp.float32),
               pltpu.VMEM((8, ch), jnp.float32)]
    scratch += [pltpu.VMEM((npad,), jnp.float32)] * naccs
    scratch += [pltpu.SemaphoreType.DMA]

    @functools.partial(
        pl.kernel, out_type=out_type, mesh=_sc_mesh(),
        scratch_types=scratch,
        compiler_params=pltpu.CompilerParams(needs_layout_passes=False))
    def k(*refs):
        tab1_hbm, tab2_hbm, rT_hbm, idx_hbm, z_hbm = refs[:5]
        valT_hbm = refs[5]
        p_hbm = refs[6:6 + naccs]
        tab1_v, tab2_v, idx_v, r_v, v_v = refs[6 + naccs:11 + naccs]
        accs = refs[11 + naccs:11 + 2 * naccs]

        cid = lax.axis_index("c")
        sid = lax.axis_index("s")
        wid = sid * _NC + cid
        pltpu.sync_copy(tab1_hbm, tab1_v)
        pltpu.sync_copy(tab2_hbm, tab2_v)
        for a in accs:
            pltpu.sync_copy(z_hbm, a)
        one16 = jnp.ones((16,), jnp.float32)

        def body(j, carry):
            c = wid + j * _NW

            @pl.when(c < chunks)
            def _():
                pltpu.sync_copy(idx_hbm.at[pl.ds(c * ch, ch)], idx_v)
                pltpu.sync_copy(rT_hbm.at[:, pl.ds(c * ch, ch)], r_v)
                for off in range(0, ch, 16):
                    if True:
                        d = idx_v[pl.ds(off, 16)]
                        t1 = plsc.load_gather(tab1_v, [d])
                        t2 = plsc.load_gather(tab2_v, [d])
                        r1 = r_v[0, pl.ds(off, 16)]
                        r2 = r_v[1, pl.ds(off, 16)]
                        if mode == 'stats':
                            v1 = _lrelu(t1 + r1)
                            v2 = _lrelu(t2 + r2)
                        else:
                            v1 = jnp.exp(r1 - t1)
                            v2 = jnp.exp(r2 - t2)
                        v_v[0, pl.ds(off, 16)] = v1
                        v_v[1, pl.ds(off, 16)] = v2
                        plsc.addupdate_scatter(accs[0], [d], v1)
                        plsc.addupdate_scatter(accs[1], [d], v2)
                        if mode == 'stats':
                            plsc.addupdate_scatter(accs[2], [d], one16)
                pltpu.sync_copy(v_v, valT_hbm.at[:, pl.ds(c * ch, ch)])

            return carry

        lax.fori_loop(0, nk, body, 0)
        for a, p in zip(accs, p_hbm):
            pltpu.sync_copy(a, p.at[wid])

    outs = k(tab1, tab2, rT, idxflat, zeros)
    return outs[0], outs[1:]


def _pad_rows(x, m=_CH):
    n = x.shape[0]
    p = (-n) % m
    return x if p == 0 else jnp.pad(x, ((0, p),) + ((0, 0),) * (x.ndim - 1))


def _pad_flat(idx, fill, m=_CH):
    n = idx.shape[0]
    p = (-n) % m
    if p:
        idx = jnp.concatenate([idx, jnp.full((p,), fill, jnp.int32)])
    return idx


def _blk(n, target):
    if n <= target:
        return n
    best = None
    b = 128
    while b <= target:
        if n % b == 0:
            best = b
        b += 128
    if best is None:
        k = -(-n // target)
        while n % k:
            k += 1
        best = n // k
    return best


def _gather_rows(tab, idx):
    return jnp.take(tab, idx, axis=0)


def _segsum(x, seg, n):
    return jax.ops.segment_sum(x, seg, num_segments=n)


def _softmax_stage(t, a8, seg, nseg):
    suma = _segsum(a8, seg, nseg)
    cnt = jnp.maximum(suma[:, 2:3], 1.0)
    amean = suma[:, 0:2] / cnt
    ex = jnp.exp(a8[:, 0:2] - amean[seg])
    den = _segsum(jnp.pad(ex, ((0, 0), (0, 6))), seg, nseg)
    exw = jnp.concatenate([jnp.repeat(ex[:, 0:1], D, 1),
                           jnp.repeat(ex[:, 1:2], D, 1)], axis=1)
    num = _segsum(exw * t, seg, nseg)
    return num, den


def _atom_stage(pk, ef, src3d, dstf, dst3d, p12, q2flat, h12, npad, molpk,
                eblk, nblk):
    gp = _sc_gather(p12, src3d, 2 * D)
    t, rT = _edge_call(ef, gp, pk['We'], pk['be'], pk['W2cat'], pk['wa2T'],
                       pk['Watblk'], pk['batcat'], eblk)
    aT, sparts = _sc_narrow(q2flat[0], q2flat[1], rT, dstf, npad, 'stats')
    am1, am2 = _amean_call(sparts, _blk(npad, 2048))
    exT, den = _sc_narrow(am1.reshape(-1), am2.reshape(-1), aT, dstf, npad,
                          'ex')
    ext = _scale_call(t, exT, eblk)
    (num,) = _sc_scatter([ext], dst3d, npad)
    return _atomctx_call(num, den, h12, pk['gru'], molpk['wm2T'],
                         molpk['Watblk'], molpk['batcat'], nblk)


def _mol_stage(molpk, h12, rmT, tm12, n2gf, n2g3d, nsegpad, nblk, sblk):
    (sparts,) = _sc_scatter([h12], n2g3d, nsegpad)
    s12, qm8 = _molqm_call(sparts, molpk['wm1blk'], molpk['qb'], sblk)
    aT, stparts = _sc_narrow(qm8[:, 0], qm8[:, 1], rmT, n2gf, nsegpad,
                             'stats')
    am1, am2 = _amean_call(stparts, sblk)
    exT, den = _sc_narrow(am1.reshape(-1), am2.reshape(-1), aT, n2gf,
                          nsegpad, 'ex')
    ext = _scale_call(tm12, exT, nblk)
    (num,) = _sc_scatter([ext], n2g3d, nsegpad)
    return _molctx_call(num, den, s12, molpk['gru'], sblk)


def kernel(params, origin_node, origin_edge, origin_edge_index, frag_node,
           frag_edge, frag_edge_index, frag_node2graph, motif_node,
           motif_edge, motif_edge_index, motif_node2graph):
    V = frag_node.shape[0]
    E = frag_edge.shape[0]
    NF = motif_node.shape[0]
    EM = motif_edge.shape[0]
    G = NUM_GRAPHS
    Vp = V + ((-V) % _CH)
    EMp = EM + ((-EM) % _CH)
    NFp = NF + ((-NF) % _CH)
    Gp = G + ((-G) % _CH)

    fsrc = _pad_flat(jnp.asarray(frag_edge_index[0], jnp.int32), 0)
    fdst = _pad_flat(jnp.asarray(frag_edge_index[1], jnp.int32), Vp - 1)
    msrc = _pad_flat(jnp.asarray(motif_edge_index[0], jnp.int32), 0)
    mdst = _pad_flat(jnp.asarray(motif_edge_index[1], jnp.int32), NFp - 1)
    fn2g = _pad_flat(jnp.asarray(frag_node2graph, jnp.int32), NFp - 1)
    mn2g = _pad_flat(jnp.asarray(motif_node2graph, jnp.int32), Gp - 1)
    r3 = lambda x: x.reshape(-1, _CH // _IB, _IB)
    r2 = lambda x: x.reshape(-1, 2, _IB)
    fsrc3d = r3(fsrc)
    fdst3d = r2(fdst)
    msrc3d = r3(msrc)
    mdst3d = r2(mdst)
    fn2g3d = r2(fn2g)
    mn2g3d = r2(mn2g)

    fh = params['frag_heads']
    jh = params['junc_heads']

    fa = _pack_atom([h['atom'] for h in fh])
    fa['We'] = params['emb_fe']['W']
    fa['be'] = params['emb_fe']['b'][None, :]
    fm = _pack_mol([h['mol'] for h in fh])
    ja = _pack_atom([h['atom'] for h in jh])
    ja['We'] = params['emb_me']['W']
    ja['be'] = params['emb_me']['b'][None, :]
    jm = _pack_mol([h['mol'] for h in jh])

    fn, p12, q12 = _pre_call(
        frag_node, params['emb_fn']['W'], params['emb_fn']['b'][None, :],
        fa['W1cat'], fa['b1cat'], fa['wa1cat'], fa['qb'], _blk(V, 2000))
    h12 = _pad_rows(jnp.concatenate([fn, fn], axis=1))
    q12p = _pad_rows(q12)
    q2flat = (q12p[:, 0], q12p[:, 1])
    hn12, rmT, tm12 = _atom_stage(fa, _pad_rows(frag_edge), fsrc3d, fdst,
                                  fdst3d, _pad_rows(p12), q2flat, h12, Vp, fm,
                                  _blk(fsrc.shape[0], 2048), _blk(Vp, 2048))
    sf12 = _mol_stage(fm, hn12, rmT, tm12, fn2g, fn2g3d, NFp, _blk(Vp, 2048),
                      _blk(NFp, 2560))
    gm = _gm_call(sf12, params['frag_attend']['W'],
                  params['frag_attend']['b'][None, :], _blk(NFp, 2560))

    Pcat = jnp.concatenate([h['proj']['W'] for h in jh], axis=1)
    pbcat = jnp.concatenate([h['proj']['b'] for h in jh])[None, :]
    W1blk = block_diag(*[h['atom']['ne']['W'][:D] for h in jh])
    b1cat = jnp.concatenate([h['atom']['ne']['b'] for h in jh])[None, :]
    wa1blk = jnp.pad(block_diag(*[h['atom']['al']['W'][:D] for h in jh]),
                     ((0, 0), (0, 6)))
    jqb = jnp.array([jh[0]['atom']['al']['b'][0], jh[1]['atom']['al']['b'][0],
                     1.0, 0, 0, 0, 0, 0], jnp.float32)[None, :]
    x12, pj12, qj12 = _motifpre_call(motif_node, gm[:NF], params['emb_mn']['W'],
                                     params['emb_mn']['b'][None, :], Pcat,
                                     pbcat, W1blk, b1cat, wa1blk, jqb,
                                     _blk(NF, 2500))

    pj12 = _pad_rows(pj12)
    qj12p = _pad_rows(qj12)
    qj2flat = (qj12p[:, 0], qj12p[:, 1])
    xn12, rjT, tj12 = _atom_stage(ja, _pad_rows(motif_edge), msrc3d, mdst,
                                  mdst3d, pj12, qj2flat, _pad_rows(x12), NFp,
                                  jm, _blk(EMp, 2048), _blk(NFp, 2560))
    sg12 = _mol_stage(jm, xn12, rjT, tj12, mn2g, mn2g3d, Gp, _blk(NFp, 2560),
                      Gp)

    out = _final_call(sg12, params['pred1']['W'],
                      params['pred1']['b'][None, :], params['pred2']['W'],
                      params['pred2']['b'][None, :])
    return out[:G]
